# Initial kernel scaffold; baseline (speedup 1.0000x reference)
import functools

import jax
import jax.numpy as jnp
from jax import lax
from jax.experimental import pallas as pl
from jax.experimental.pallas import tpu as pltpu
from jax.experimental.pallas import tpu_sc as plsc

L = 16
NSUB = 16
NCORE = 2
NW = NCORE * NSUB
CH = 128


def _round_up(a, b):
  return (a + b - 1) // b * b


def _fill_flat(buf, nwords, value):
  v = jnp.full((L,), value, jnp.float32)

  def body(i, _):
    buf[pl.ds(pl.multiple_of(i * L, L), L)] = v
    return 0

  lax.fori_loop(0, nwords // L, body, 0)


def _zero_rows(buf, rows, cols):
  z = jnp.zeros((L,), jnp.float32)
  per_row = cols // L

  def body(i, _):
    r = i // per_row
    c = i % per_row
    buf[r, pl.ds(pl.multiple_of(c * L, L), L)] = z
    return 0

  lax.fori_loop(0, rows * per_row, body, 0)


def _deg_call(dstp, M, CPW):
  TS = M // NSUB
  mesh = plsc.VectorSubcoreMesh(core_axis_name="c", subcore_axis_name="s")

  @functools.partial(
      pl.kernel,
      out_type=jax.ShapeDtypeStruct((NCORE, M), jnp.float32),
      mesh=mesh,
      scratch_types=[
          pltpu.VMEM_SHARED((M,), jnp.float32),
          pltpu.VMEM((CPW, CH), jnp.int32),
          pltpu.VMEM((CH,), jnp.float32),
          pltpu.VMEM((TS,), jnp.float32),
      ],
  )
  def k(dstp_hbm, deg_hbm, acc, dstv, vones, zbuf):
    c = lax.axis_index("c")
    s = lax.axis_index("s")
    base = pl.multiple_of(s * TS, 64)
    _fill_flat(zbuf, TS, 0.0)
    pltpu.sync_copy(zbuf, acc.at[pl.ds(base, TS)])
    _fill_flat(vones, CH, 1.0)
    plsc.subcore_barrier()

    w = c * NSUB + s
    pltpu.sync_copy(dstp_hbm.at[w], dstv)

    def chunk(j, _):
      pltpu.sync_copy(vones, acc.at[dstv.at[j]], add=True)
      return 0

    lax.fori_loop(0, CPW, chunk, 0)
    plsc.subcore_barrier()
    pltpu.sync_copy(acc.at[pl.ds(base, TS)], zbuf)
    pltpu.sync_copy(zbuf, deg_hbm.at[c, pl.ds(base, TS)])

  return k(dstp)


def _scale_call(degp, x, M):
  N, D = x.shape

  def body(d0_ref, d1_ref, x_ref, xs_ref, dinv_ref):
    deg = d0_ref[...] + d1_ref[...]
    dinv = lax.rsqrt(jnp.maximum(deg, 1.0))
    dinv_ref[...] = dinv
    xs_ref[...] = x_ref[...] * dinv[:N]

  return pl.pallas_call(
      body,
      out_shape=[
          jax.ShapeDtypeStruct((N, D), jnp.float32),
          jax.ShapeDtypeStruct((M, 1), jnp.float32),
      ],
  )(degp[0].reshape(M, 1), degp[1].reshape(M, 1), x)


def _prop_call(xs, srcp, dstp, M, CPW):
  N, D = xs.shape
  TS = M // NSUB
  mesh = plsc.VectorSubcoreMesh(core_axis_name="c", subcore_axis_name="s")

  @functools.partial(
      pl.kernel,
      out_type=jax.ShapeDtypeStruct((NCORE, M, D), jnp.float32),
      mesh=mesh,
      scratch_types=[
          pltpu.VMEM_SHARED((M, D), jnp.float32),
          pltpu.VMEM((CPW, CH), jnp.int32),
          pltpu.VMEM((CPW, CH), jnp.int32),
          pltpu.VMEM((CH, D), jnp.float32),
          pltpu.VMEM((CH, D), jnp.float32),
          pltpu.VMEM((64, D), jnp.float32),
          pltpu.SemaphoreType.DMA,
          pltpu.SemaphoreType.DMA,
      ],
  )
  def k(xs_hbm, srcp_hbm, dstp_hbm, pp_hbm,
        acc, srcv, dstv, rv0, rv1, zrow, sem0, sem1):
    c = lax.axis_index("c")
    s = lax.axis_index("s")
    base = pl.multiple_of(s * TS, 64)
    _zero_rows(zrow, 64, D)

    def zc(kk, _):
      pltpu.sync_copy(zrow, acc.at[pl.ds(base + kk * 64, 64)])
      return 0

    lax.fori_loop(0, TS // 64, zc, 0)
    plsc.subcore_barrier()

    w = c * NSUB + s
    pltpu.sync_copy(srcp_hbm.at[w], srcv)
    pltpu.sync_copy(dstp_hbm.at[w], dstv)

    def chunk2(t, _):
      j0 = 2 * t
      j1 = 2 * t + 1
      g0 = pltpu.async_copy(xs_hbm.at[srcv.at[j0]], rv0, sem0)
      g1 = pltpu.async_copy(xs_hbm.at[srcv.at[j1]], rv1, sem1)
      g0.wait()
      pltpu.sync_copy(rv0, acc.at[dstv.at[j0]], add=True)
      g1.wait()
      pltpu.sync_copy(rv1, acc.at[dstv.at[j1]], add=True)
      return 0

    lax.fori_loop(0, CPW // 2, chunk2, 0)
    plsc.subcore_barrier()

    def oc(kk, _):
      rbase = pl.multiple_of(base + kk * CH, 64)
      pltpu.sync_copy(acc.at[pl.ds(rbase, CH)], rv0)
      pltpu.sync_copy(rv0, pp_hbm.at[c, pl.ds(rbase, CH)])
      return 0

    lax.fori_loop(0, TS // CH, oc, 0)

  return k(xs, srcp, dstp)


def _dense_call(pp, dinv, eW1, eb1, eW2, eb2, gW1, gb1, gW2, gb2, N):
  D = pp.shape[2]
  H = eW1.shape[1]
  BR = 2000 if N % 2000 == 0 else N
  grid = (N // BR,)

  def body(p0_ref, p1_ref, dinv_ref, eW1_ref, eb1_ref, eW2_ref, eb2_ref,
           gW1_ref, gb1_ref, gW2_ref, gb2_ref, we_ref, wg_ref):
    dv = dinv_ref[...]
    p = (p0_ref[...] + p1_ref[...]) * dv

    he = jnp.maximum(
        jnp.dot(p, eW1_ref[...], preferred_element_type=jnp.float32)
        + eb1_ref[...], 0.0)
    we_ref[...] = (
        jnp.dot(he, eW2_ref[...], preferred_element_type=jnp.float32)
        + eb2_ref[...]) * dv

    hg = jnp.maximum(
        jnp.dot(p, gW1_ref[...], preferred_element_type=jnp.float32)
        + gb1_ref[...], 0.0)
    wg_ref[...] = (
        jnp.dot(hg, gW2_ref[...], preferred_element_type=jnp.float32)
        + gb2_ref[...]) * dv

  row_spec = pl.BlockSpec((BR, D), lambda i: (i, 0))
  col_spec = pl.BlockSpec((BR, 1), lambda i: (i, 0))
  w1_spec = pl.BlockSpec((D, H), lambda i: (0, 0))
  b1_spec = pl.BlockSpec((1, H), lambda i: (0, 0))
  w2_spec = pl.BlockSpec((H, 1), lambda i: (0, 0))
  b2_spec = pl.BlockSpec((1, 1), lambda i: (0, 0))

  return pl.pallas_call(
      body,
      grid=grid,
      in_specs=[row_spec, row_spec, col_spec,
                w1_spec, b1_spec, w2_spec, b2_spec,
                w1_spec, b1_spec, w2_spec, b2_spec],
      out_specs=[col_spec, col_spec],
      out_shape=[
          jax.ShapeDtypeStruct((N, 1), jnp.float32),
          jax.ShapeDtypeStruct((N, 1), jnp.float32),
      ],
  )(pp[0], pp[1], dinv,
    eW1, eb1.reshape(1, H), eW2, eb2.reshape(1, 1),
    gW1, gb1.reshape(1, H), gW2, gb2.reshape(1, 1))


def _final_call(we, wg, srcp, dstp, dinv, M, CPW):
  TS = M // NSUB
  WPT = NW // NSUB
  mesh = plsc.VectorSubcoreMesh(
      core_axis_name="c", subcore_axis_name="s", num_cores=1)

  @functools.partial(
      pl.kernel,
      out_type=[
          jax.ShapeDtypeStruct((M,), jnp.float32),
          jax.ShapeDtypeStruct((M,), jnp.float32),
      ],
      mesh=mesh,
      scratch_types=[
          pltpu.VMEM_SHARED((M,), jnp.float32),
          pltpu.VMEM_SHARED((M,), jnp.float32),
          pltpu.VMEM((CPW, CH), jnp.int32),
          pltpu.VMEM((CPW, CH), jnp.int32),
          pltpu.VMEM((CH,), jnp.float32),
          pltpu.VMEM((CH,), jnp.float32),
          pltpu.VMEM((TS,), jnp.float32),
          pltpu.VMEM((TS,), jnp.float32),
          pltpu.SemaphoreType.DMA,
          pltpu.SemaphoreType.DMA,
      ],
  )
  def k(we_hbm, wg_hbm, srcp_hbm, dstp_hbm, dinv_hbm, s_hbm, y_hbm,
        accS, accY, srcv, dstv, vbS, vbY, tb, db, semS, semY):
    s = lax.axis_index("s")
    base = pl.multiple_of(s * TS, 64)
    _fill_flat(tb, TS, 0.0)
    pltpu.sync_copy(tb, accS.at[pl.ds(base, TS)])
    pltpu.sync_copy(tb, accY.at[pl.ds(base, TS)])
    plsc.subcore_barrier()

    for r in range(WPT):
      w = s * WPT + r
      pltpu.sync_copy(srcp_hbm.at[w], srcv)
      pltpu.sync_copy(dstp_hbm.at[w], dstv)

      def chunk(j, _):
        gS = pltpu.async_copy(we_hbm.at[srcv.at[j]], vbS, semS)
        gY = pltpu.async_copy(wg_hbm.at[srcv.at[j]], vbY, semY)
        gS.wait()
        pltpu.sync_copy(vbS, accS.at[dstv.at[j]], add=True)
        gY.wait()
        pltpu.sync_copy(vbY, accY.at[dstv.at[j]], add=True)
        return 0

      lax.fori_loop(0, CPW, chunk, 0)

    plsc.subcore_barrier()
    pltpu.sync_copy(dinv_hbm.at[pl.ds(base, TS)], db)

    def mul(i, _):
      o = pl.multiple_of(i * L, L)
      tb[pl.ds(o, L)] = tb[pl.ds(o, L)] * db[pl.ds(o, L)]
      return 0

    pltpu.sync_copy(accS.at[pl.ds(base, TS)], tb)
    lax.fori_loop(0, TS // L, mul, 0)
    pltpu.sync_copy(tb, s_hbm.at[pl.ds(base, TS)])

    pltpu.sync_copy(accY.at[pl.ds(base, TS)], tb)
    lax.fori_loop(0, TS // L, mul, 0)
    pltpu.sync_copy(tb, y_hbm.at[pl.ds(base, TS)])

  return k(we, wg, srcp, dstp, dinv)


def kernel(adj, x, est_W1, est_b1, est_W2, est_b2,
           gnn_W1, gnn_b1, gnn_W2, gnn_b2):
  N, D = x.shape
  E = adj.shape[1]

  src = adj[0].astype(jnp.int32)
  dst = adj[1].astype(jnp.int32)

  EPW = -(-E // NW)
  CPW = _round_up(-(-EPW // CH), 2)
  EP = CPW * CH
  M = _round_up(N + 8, NSUB * 64)

  def padw(a, fill):
    a = jnp.pad(a, (0, NW * EPW - E), constant_values=fill).reshape(NW, EPW)
    a = jnp.pad(a, ((0, 0), (0, EP - EPW)), constant_values=fill)
    return a.reshape(NW, CPW, CH)

  srcp = padw(src, 0)
  dstp = padw(dst, N)

  degp = _deg_call(dstp, M, CPW)
  xs, dinv = _scale_call(degp, x, M)
  pp = _prop_call(xs, srcp, dstp, M, CPW)
  we, wg = _dense_call(pp, dinv, est_W1, est_b1, est_W2, est_b2,
                       gnn_W1, gnn_b1, gnn_W2, gnn_b2, N)
  sflat, yflat = _final_call(we.reshape(N), wg.reshape(N),
                             srcp, dstp, dinv.reshape(M), M, CPW)

  s = sflat[:N].reshape(N, 1)
  y = yflat[:N].reshape(N, 1)
  return (y, s)


# speedup vs baseline: 17.4946x; 17.4946x over previous
import functools

import jax
import jax.numpy as jnp
from jax import lax
from jax.experimental import pallas as pl
from jax.experimental.pallas import tpu as pltpu
from jax.experimental.pallas import tpu_sc as plsc

L = 16
NSUB = 16
NCORE = 2
NW = NCORE * NSUB
CH = 128


def _round_up(a, b):
  return (a + b - 1) // b * b


def _fill_flat(buf, nwords, value):
  v = jnp.full((L,), value, jnp.float32)

  def body(i, _):
    buf[pl.ds(pl.multiple_of(i * L, L), L)] = v
    return 0

  lax.fori_loop(0, nwords // L, body, 0)


def _zero_rows(buf, rows, cols):
  z = jnp.zeros((L,), jnp.float32)
  per_row = cols // L

  def body(i, _):
    r = i // per_row
    c = i % per_row
    buf[r, pl.ds(pl.multiple_of(c * L, L), L)] = z
    return 0

  lax.fori_loop(0, rows * per_row, body, 0)


def _deg_call(dstp, M, CPW):
  TS = M // NSUB
  mesh = plsc.VectorSubcoreMesh(core_axis_name="c", subcore_axis_name="s")

  @functools.partial(
      pl.kernel,
      out_type=jax.ShapeDtypeStruct((NCORE, M), jnp.float32),
      mesh=mesh,
      scratch_types=[
          pltpu.VMEM_SHARED((M,), jnp.float32),
          pltpu.VMEM((CPW, CH), jnp.int32),
          pltpu.VMEM((CH,), jnp.float32),
          pltpu.VMEM((TS,), jnp.float32),
      ],
  )
  def k(dstp_hbm, deg_hbm, acc, dstv, vones, zbuf):
    c = lax.axis_index("c")
    s = lax.axis_index("s")
    base = pl.multiple_of(s * TS, 64)
    _fill_flat(zbuf, TS, 0.0)
    pltpu.sync_copy(zbuf, acc.at[pl.ds(base, TS)])
    _fill_flat(vones, CH, 1.0)
    plsc.subcore_barrier()

    w = c * NSUB + s
    pltpu.sync_copy(dstp_hbm.at[w], dstv)

    def chunk(j, _):
      pltpu.sync_copy(vones, acc.at[dstv.at[j]], add=True)
      return 0

    lax.fori_loop(0, CPW, chunk, 0)
    plsc.subcore_barrier()
    pltpu.sync_copy(acc.at[pl.ds(base, TS)], zbuf)
    pltpu.sync_copy(zbuf, deg_hbm.at[c, pl.ds(base, TS)])

  return k(dstp)


def _scale_call(degp, x, M):
  N, D = x.shape

  def body(d0_ref, d1_ref, x_ref, xs_ref, dinv_ref):
    deg = d0_ref[...] + d1_ref[...]
    dinv = lax.rsqrt(jnp.maximum(deg, 1.0))
    dinv_ref[...] = dinv
    xs_ref[...] = x_ref[...] * dinv[:N]

  return pl.pallas_call(
      body,
      out_shape=[
          jax.ShapeDtypeStruct((N, D), jnp.float32),
          jax.ShapeDtypeStruct((M, 1), jnp.float32),
      ],
  )(degp[0].reshape(M, 1), degp[1].reshape(M, 1), x)


def _prop_call(xs, srcp, dstp, M, CPW):
  N, D = xs.shape
  TS = M // NSUB
  HCP = CPW // 2
  mesh = plsc.VectorSubcoreMesh(core_axis_name="c", subcore_axis_name="s")

  @functools.partial(
      pl.kernel,
      out_type=jax.ShapeDtypeStruct((NCORE, M, D), jnp.float32),
      mesh=mesh,
      scratch_types=[
          pltpu.VMEM_SHARED((M, D), jnp.float32),
          pltpu.VMEM((HCP, CH), jnp.int32),
          pltpu.VMEM((HCP, CH), jnp.int32),
          pltpu.VMEM((CH, D), jnp.float32),
          pltpu.VMEM((CH, D), jnp.float32),
          pltpu.SemaphoreType.DMA,
          pltpu.SemaphoreType.DMA,
      ],
  )
  def k(xs_hbm, srcp_hbm, dstp_hbm, pp_hbm,
        acc, srcv, dstv, rv0, rv1, sem0, sem1):
    c = lax.axis_index("c")
    s = lax.axis_index("s")
    base = pl.multiple_of(s * TS, 64)
    _zero_rows(rv0, CH, D)

    def zc(kk, _):
      pltpu.sync_copy(rv0, acc.at[pl.ds(base + kk * CH, CH)])
      return 0

    lax.fori_loop(0, TS // CH, zc, 0)
    plsc.subcore_barrier()

    w = c * NSUB + s

    def chunk2(t, _):
      j0 = 2 * t
      j1 = 2 * t + 1
      g0 = pltpu.async_copy(xs_hbm.at[srcv.at[j0]], rv0, sem0)
      g1 = pltpu.async_copy(xs_hbm.at[srcv.at[j1]], rv1, sem1)
      g0.wait()
      pltpu.sync_copy(rv0, acc.at[dstv.at[j0]], add=True)
      g1.wait()
      pltpu.sync_copy(rv1, acc.at[dstv.at[j1]], add=True)
      return 0

    for h in range(2):
      pltpu.sync_copy(srcp_hbm.at[w, pl.ds(h * HCP, HCP)], srcv)
      pltpu.sync_copy(dstp_hbm.at[w, pl.ds(h * HCP, HCP)], dstv)
      lax.fori_loop(0, HCP // 2, chunk2, 0)

    plsc.subcore_barrier()

    def oc(kk, _):
      rbase = pl.multiple_of(base + kk * CH, 64)
      pltpu.sync_copy(acc.at[pl.ds(rbase, CH)], rv0)
      pltpu.sync_copy(rv0, pp_hbm.at[c, pl.ds(rbase, CH)])
      return 0

    lax.fori_loop(0, TS // CH, oc, 0)

  return k(xs, srcp, dstp)


def _dense_call(pp, dinv, eW1, eb1, eW2, eb2, gW1, gb1, gW2, gb2, N):
  D = pp.shape[2]
  H = eW1.shape[1]
  BR = 2000 if N % 2000 == 0 else N
  grid = (N // BR,)

  def body(p0_ref, p1_ref, dinv_ref, eW1_ref, eb1_ref, eW2_ref, eb2_ref,
           gW1_ref, gb1_ref, gW2_ref, gb2_ref, we_ref, wg_ref):
    dv = dinv_ref[...]
    p = (p0_ref[...] + p1_ref[...]) * dv

    he = jnp.maximum(
        jnp.dot(p, eW1_ref[...], preferred_element_type=jnp.float32)
        + eb1_ref[...], 0.0)
    we_ref[...] = (
        jnp.dot(he, eW2_ref[...], preferred_element_type=jnp.float32)
        + eb2_ref[...]) * dv

    hg = jnp.maximum(
        jnp.dot(p, gW1_ref[...], preferred_element_type=jnp.float32)
        + gb1_ref[...], 0.0)
    wg_ref[...] = (
        jnp.dot(hg, gW2_ref[...], preferred_element_type=jnp.float32)
        + gb2_ref[...]) * dv

  row_spec = pl.BlockSpec((BR, D), lambda i: (i, 0))
  col_spec = pl.BlockSpec((BR, 1), lambda i: (i, 0))
  w1_spec = pl.BlockSpec((D, H), lambda i: (0, 0))
  b1_spec = pl.BlockSpec((1, H), lambda i: (0, 0))
  w2_spec = pl.BlockSpec((H, 1), lambda i: (0, 0))
  b2_spec = pl.BlockSpec((1, 1), lambda i: (0, 0))

  return pl.pallas_call(
      body,
      grid=grid,
      in_specs=[row_spec, row_spec, col_spec,
                w1_spec, b1_spec, w2_spec, b2_spec,
                w1_spec, b1_spec, w2_spec, b2_spec],
      out_specs=[col_spec, col_spec],
      out_shape=[
          jax.ShapeDtypeStruct((N, 1), jnp.float32),
          jax.ShapeDtypeStruct((N, 1), jnp.float32),
      ],
  )(pp[0], pp[1], dinv,
    eW1, eb1.reshape(1, H), eW2, eb2.reshape(1, 1),
    gW1, gb1.reshape(1, H), gW2, gb2.reshape(1, 1))


def _final_call(we, wg, srcp, dstp, dinv, M, CPW):
  TS = M // NSUB
  WPT = NW // NSUB
  mesh = plsc.VectorSubcoreMesh(
      core_axis_name="c", subcore_axis_name="s", num_cores=1)

  @functools.partial(
      pl.kernel,
      out_type=[
          jax.ShapeDtypeStruct((M,), jnp.float32),
          jax.ShapeDtypeStruct((M,), jnp.float32),
      ],
      mesh=mesh,
      scratch_types=[
          pltpu.VMEM_SHARED((M,), jnp.float32),
          pltpu.VMEM_SHARED((M,), jnp.float32),
          pltpu.VMEM((CPW, CH), jnp.int32),
          pltpu.VMEM((CPW, CH), jnp.int32),
          pltpu.VMEM((CH,), jnp.float32),
          pltpu.VMEM((CH,), jnp.float32),
          pltpu.VMEM((TS,), jnp.float32),
          pltpu.VMEM((TS,), jnp.float32),
          pltpu.SemaphoreType.DMA,
          pltpu.SemaphoreType.DMA,
      ],
  )
  def k(we_hbm, wg_hbm, srcp_hbm, dstp_hbm, dinv_hbm, s_hbm, y_hbm,
        accS, accY, srcv, dstv, vbS, vbY, tb, db, semS, semY):
    s = lax.axis_index("s")
    base = pl.multiple_of(s * TS, 64)
    _fill_flat(tb, TS, 0.0)
    pltpu.sync_copy(tb, accS.at[pl.ds(base, TS)])
    pltpu.sync_copy(tb, accY.at[pl.ds(base, TS)])
    plsc.subcore_barrier()

    for r in range(WPT):
      w = s * WPT + r
      pltpu.sync_copy(srcp_hbm.at[w], srcv)
      pltpu.sync_copy(dstp_hbm.at[w], dstv)

      def chunk(j, _):
        gS = pltpu.async_copy(we_hbm.at[srcv.at[j]], vbS, semS)
        gY = pltpu.async_copy(wg_hbm.at[srcv.at[j]], vbY, semY)
        gS.wait()
        pltpu.sync_copy(vbS, accS.at[dstv.at[j]], add=True)
        gY.wait()
        pltpu.sync_copy(vbY, accY.at[dstv.at[j]], add=True)
        return 0

      lax.fori_loop(0, CPW, chunk, 0)

    plsc.subcore_barrier()
    pltpu.sync_copy(dinv_hbm.at[pl.ds(base, TS)], db)

    def mul(i, _):
      o = pl.multiple_of(i * L, L)
      tb[pl.ds(o, L)] = tb[pl.ds(o, L)] * db[pl.ds(o, L)]
      return 0

    pltpu.sync_copy(accS.at[pl.ds(base, TS)], tb)
    lax.fori_loop(0, TS // L, mul, 0)
    pltpu.sync_copy(tb, s_hbm.at[pl.ds(base, TS)])

    pltpu.sync_copy(accY.at[pl.ds(base, TS)], tb)
    lax.fori_loop(0, TS // L, mul, 0)
    pltpu.sync_copy(tb, y_hbm.at[pl.ds(base, TS)])

  return k(we, wg, srcp, dstp, dinv)


def kernel(adj, x, est_W1, est_b1, est_W2, est_b2,
           gnn_W1, gnn_b1, gnn_W2, gnn_b2):
  N, D = x.shape
  E = adj.shape[1]

  src = adj[0].astype(jnp.int32)
  dst = adj[1].astype(jnp.int32)

  EPW = -(-E // NW)
  CPW = _round_up(-(-EPW // CH), 4)
  EP = CPW * CH
  M = _round_up(N + 8, NSUB * 64)

  def padw(a, fill):
    a = jnp.pad(a, (0, NW * EPW - E), constant_values=fill).reshape(NW, EPW)
    a = jnp.pad(a, ((0, 0), (0, EP - EPW)), constant_values=fill)
    return a.reshape(NW, CPW, CH)

  srcp = padw(src, 0)
  dstp = padw(dst, N)

  degp = _deg_call(dstp, M, CPW)
  xs, dinv = _scale_call(degp, x, M)
  pp = _prop_call(xs, srcp, dstp, M, CPW)
  we, wg = _dense_call(pp, dinv, est_W1, est_b1, est_W2, est_b2,
                       gnn_W1, gnn_b1, gnn_W2, gnn_b2, N)
  sflat, yflat = _final_call(we.reshape(N), wg.reshape(N),
                             srcp, dstp, dinv.reshape(M), M, CPW)

  s = sflat[:N].reshape(N, 1)
  y = yflat[:N].reshape(N, 1)
  return (y, s)


# speedup vs baseline: 19.8470x; 1.1345x over previous
import functools

import jax
import jax.numpy as jnp
from jax import lax
from jax.experimental import pallas as pl
from jax.experimental.pallas import tpu as pltpu
from jax.experimental.pallas import tpu_sc as plsc

L = 16
NSUB = 16
NCORE = 2
NW = NCORE * NSUB
CH = 128


def _round_up(a, b):
  return (a + b - 1) // b * b


def _fill_flat(buf, nwords, value):
  v = jnp.full((L,), value, jnp.float32)

  def body(i, _):
    buf[pl.ds(pl.multiple_of(i * L, L), L)] = v
    return 0

  lax.fori_loop(0, nwords // L, body, 0)


def _zero_rows(buf, rows, cols):
  z = jnp.zeros((L,), jnp.float32)
  per_row = cols // L

  def body(i, _):
    r = i // per_row
    c = i % per_row
    buf[r, pl.ds(pl.multiple_of(c * L, L), L)] = z
    return 0

  lax.fori_loop(0, rows * per_row, body, 0)


def _deg_call(dstp, M, CPW):
  TS = M // NSUB
  mesh = plsc.VectorSubcoreMesh(core_axis_name="c", subcore_axis_name="s")

  @functools.partial(
      pl.kernel,
      out_type=jax.ShapeDtypeStruct((NCORE, M), jnp.float32),
      mesh=mesh,
      scratch_types=[
          pltpu.VMEM_SHARED((M,), jnp.float32),
          pltpu.VMEM((CPW, CH), jnp.int32),
          pltpu.VMEM((CH,), jnp.float32),
          pltpu.VMEM((TS,), jnp.float32),
      ],
  )
  def k(dstp_hbm, deg_hbm, acc, dstv, vones, zbuf):
    c = lax.axis_index("c")
    s = lax.axis_index("s")
    base = pl.multiple_of(s * TS, 64)
    _fill_flat(zbuf, TS, 0.0)
    pltpu.sync_copy(zbuf, acc.at[pl.ds(base, TS)])
    _fill_flat(vones, CH, 1.0)
    plsc.subcore_barrier()

    w = c * NSUB + s
    pltpu.sync_copy(dstp_hbm.at[w], dstv)

    def chunk(j, _):
      pltpu.sync_copy(vones, acc.at[dstv.at[j]], add=True)
      return 0

    lax.fori_loop(0, CPW, chunk, 0)
    plsc.subcore_barrier()
    pltpu.sync_copy(acc.at[pl.ds(base, TS)], zbuf)
    pltpu.sync_copy(zbuf, deg_hbm.at[c, pl.ds(base, TS)])

  return k(dstp)


def _scale_call(degp, x, M):
  N, D = x.shape

  def body(d0_ref, d1_ref, x_ref, xs_ref, dinv_ref):
    deg = d0_ref[...] + d1_ref[...]
    dinv = lax.rsqrt(jnp.maximum(deg, 1.0))
    dinv_ref[...] = dinv
    xs_ref[...] = x_ref[...] * dinv[:N]

  return pl.pallas_call(
      body,
      out_shape=[
          jax.ShapeDtypeStruct((N, D), jnp.float32),
          jax.ShapeDtypeStruct((M, 1), jnp.float32),
      ],
  )(degp[0].reshape(M, 1), degp[1].reshape(M, 1), x)


def _prop_call(xs, srcp, dstp, M, CPW):
  N, D = xs.shape
  TS = M // NSUB
  HCP = CPW // 2
  mesh = plsc.VectorSubcoreMesh(core_axis_name="c", subcore_axis_name="s")

  @functools.partial(
      pl.kernel,
      out_type=jax.ShapeDtypeStruct((NCORE, M, D), jnp.float32),
      mesh=mesh,
      scratch_types=[
          pltpu.VMEM_SHARED((M, D), jnp.float32),
          pltpu.VMEM((HCP, CH), jnp.int32),
          pltpu.VMEM((HCP, CH), jnp.int32),
          pltpu.VMEM((CH, D), jnp.float32),
          pltpu.VMEM((CH, D), jnp.float32),
          pltpu.SemaphoreType.DMA,
          pltpu.SemaphoreType.DMA,
          pltpu.SemaphoreType.DMA,
          pltpu.SemaphoreType.DMA,
      ],
  )
  def k(xs_hbm, srcp_hbm, dstp_hbm, pp_hbm,
        acc, srcv, dstv, rv0, rv1, gs0, gs1, ss0, ss1):
    c = lax.axis_index("c")
    s = lax.axis_index("s")
    base = pl.multiple_of(s * TS, 64)
    _zero_rows(rv0, CH, D)

    def zc(kk, _):
      pltpu.sync_copy(rv0, acc.at[pl.ds(base + kk * CH, CH)])
      return 0

    lax.fori_loop(0, TS // CH, zc, 0)
    plsc.subcore_barrier()

    w = c * NSUB + s
    NT = HCP // 2

    def wait_g(rv, gs):
      pltpu.make_async_copy(xs_hbm.at[srcv.at[0]], rv, gs).wait()

    def wait_s(rv, ss):
      pltpu.make_async_copy(rv, acc.at[dstv.at[0]], ss).wait()

    def chunk2(t, _):
      wait_g(rv0, gs0)
      pltpu.async_copy(rv0, acc.at[dstv.at[2 * t]], ss0, add=True)
      wait_g(rv1, gs1)
      pltpu.async_copy(rv1, acc.at[dstv.at[2 * t + 1]], ss1, add=True)

      @pl.when(t + 1 < NT)
      def _():
        wait_s(rv0, ss0)
        pltpu.async_copy(xs_hbm.at[srcv.at[2 * t + 2]], rv0, gs0)
        wait_s(rv1, ss1)
        pltpu.async_copy(xs_hbm.at[srcv.at[2 * t + 3]], rv1, gs1)

      @pl.when(t + 1 == NT)
      def _():
        wait_s(rv0, ss0)
        wait_s(rv1, ss1)

      return 0

    for h in range(2):
      pltpu.sync_copy(srcp_hbm.at[w, pl.ds(h * HCP, HCP)], srcv)
      pltpu.sync_copy(dstp_hbm.at[w, pl.ds(h * HCP, HCP)], dstv)
      pltpu.async_copy(xs_hbm.at[srcv.at[0]], rv0, gs0)
      pltpu.async_copy(xs_hbm.at[srcv.at[1]], rv1, gs1)
      lax.fori_loop(0, NT, chunk2, 0)

    plsc.subcore_barrier()

    def oc(kk, _):
      rbase = pl.multiple_of(base + kk * CH, 64)
      pltpu.sync_copy(acc.at[pl.ds(rbase, CH)], rv0)
      pltpu.sync_copy(rv0, pp_hbm.at[c, pl.ds(rbase, CH)])
      return 0

    lax.fori_loop(0, TS // CH, oc, 0)

  return k(xs, srcp, dstp)


def _dense_call(pp, dinv, eW1, eb1, eW2, eb2, gW1, gb1, gW2, gb2, N):
  D = pp.shape[2]
  H = eW1.shape[1]
  BR = 2000 if N % 2000 == 0 else N
  grid = (N // BR,)

  def body(p0_ref, p1_ref, dinv_ref, eW1_ref, eb1_ref, eW2_ref, eb2_ref,
           gW1_ref, gb1_ref, gW2_ref, gb2_ref, we_ref, wg_ref):
    dv = dinv_ref[...]
    p = (p0_ref[...] + p1_ref[...]) * dv

    he = jnp.maximum(
        jnp.dot(p, eW1_ref[...], preferred_element_type=jnp.float32)
        + eb1_ref[...], 0.0)
    we_ref[...] = (
        jnp.dot(he, eW2_ref[...], preferred_element_type=jnp.float32)
        + eb2_ref[...]) * dv

    hg = jnp.maximum(
        jnp.dot(p, gW1_ref[...], preferred_element_type=jnp.float32)
        + gb1_ref[...], 0.0)
    wg_ref[...] = (
        jnp.dot(hg, gW2_ref[...], preferred_element_type=jnp.float32)
        + gb2_ref[...]) * dv

  row_spec = pl.BlockSpec((BR, D), lambda i: (i, 0))
  col_spec = pl.BlockSpec((BR, 1), lambda i: (i, 0))
  w1_spec = pl.BlockSpec((D, H), lambda i: (0, 0))
  b1_spec = pl.BlockSpec((1, H), lambda i: (0, 0))
  w2_spec = pl.BlockSpec((H, 1), lambda i: (0, 0))
  b2_spec = pl.BlockSpec((1, 1), lambda i: (0, 0))

  return pl.pallas_call(
      body,
      grid=grid,
      in_specs=[row_spec, row_spec, col_spec,
                w1_spec, b1_spec, w2_spec, b2_spec,
                w1_spec, b1_spec, w2_spec, b2_spec],
      out_specs=[col_spec, col_spec],
      out_shape=[
          jax.ShapeDtypeStruct((N, 1), jnp.float32),
          jax.ShapeDtypeStruct((N, 1), jnp.float32),
      ],
  )(pp[0], pp[1], dinv,
    eW1, eb1.reshape(1, H), eW2, eb2.reshape(1, 1),
    gW1, gb1.reshape(1, H), gW2, gb2.reshape(1, 1))


def _final_call(we, wg, srcp, dstp, dinv, M, CPW):
  TS = M // NSUB
  mesh = plsc.VectorSubcoreMesh(core_axis_name="c", subcore_axis_name="s")

  @functools.partial(
      pl.kernel,
      out_type=[
          jax.ShapeDtypeStruct((M,), jnp.float32),
          jax.ShapeDtypeStruct((M,), jnp.float32),
      ],
      mesh=mesh,
      scratch_types=[
          pltpu.VMEM_SHARED((M,), jnp.float32),
          pltpu.VMEM(((NW // NSUB) * CPW, CH), jnp.int32),
          pltpu.VMEM(((NW // NSUB) * CPW, CH), jnp.int32),
          pltpu.VMEM((CH,), jnp.float32),
          pltpu.VMEM((CH,), jnp.float32),
          pltpu.VMEM((TS,), jnp.float32),
          pltpu.VMEM((TS,), jnp.float32),
          pltpu.SemaphoreType.DMA,
          pltpu.SemaphoreType.DMA,
          pltpu.SemaphoreType.DMA,
          pltpu.SemaphoreType.DMA,
      ],
  )
  def k(we_hbm, wg_hbm, srcp_hbm, dstp_hbm, dinv_hbm, s_hbm, y_hbm,
        acc, srcv, dstv, vb0, vb1, tb, db, gs0, gs1, ss0, ss1):
    c = lax.axis_index("c")
    s = lax.axis_index("s")
    base = pl.multiple_of(s * TS, 64)
    WPT = NW // NSUB
    NT = (WPT * CPW) // 2
    sv = srcv
    dv = dstv

    def run(w_hbm, out_hbm):
      _fill_flat(tb, TS, 0.0)
      pltpu.sync_copy(tb, acc.at[pl.ds(base, TS)])
      plsc.subcore_barrier()

      for r in range(WPT):
        pltpu.sync_copy(srcp_hbm.at[s * WPT + r],
                        srcv.at[pl.ds(r * CPW, CPW)])
        pltpu.sync_copy(dstp_hbm.at[s * WPT + r],
                        dstv.at[pl.ds(r * CPW, CPW)])

      def wait_g(vb, gs):
        pltpu.make_async_copy(w_hbm.at[sv.at[0]], vb, gs).wait()

      def wait_s(vb, ss):
        pltpu.make_async_copy(vb, acc.at[dv.at[0]], ss).wait()

      def chunk2(t, _):
        wait_g(vb0, gs0)
        pltpu.async_copy(vb0, acc.at[dv.at[2 * t]], ss0, add=True)
        wait_g(vb1, gs1)
        pltpu.async_copy(vb1, acc.at[dv.at[2 * t + 1]], ss1, add=True)

        @pl.when(t + 1 < NT)
        def _():
          wait_s(vb0, ss0)
          pltpu.async_copy(w_hbm.at[sv.at[2 * t + 2]], vb0, gs0)
          wait_s(vb1, ss1)
          pltpu.async_copy(w_hbm.at[sv.at[2 * t + 3]], vb1, gs1)

        @pl.when(t + 1 == NT)
        def _():
          wait_s(vb0, ss0)
          wait_s(vb1, ss1)

        return 0

      pltpu.async_copy(w_hbm.at[sv.at[0]], vb0, gs0)
      pltpu.async_copy(w_hbm.at[sv.at[1]], vb1, gs1)
      lax.fori_loop(0, NT, chunk2, 0)

      plsc.subcore_barrier()
      pltpu.sync_copy(dinv_hbm.at[pl.ds(base, TS)], db)
      pltpu.sync_copy(acc.at[pl.ds(base, TS)], tb)

      def mul(i, _):
        o = pl.multiple_of(i * L, L)
        tb[pl.ds(o, L)] = tb[pl.ds(o, L)] * db[pl.ds(o, L)]
        return 0

      lax.fori_loop(0, TS // L, mul, 0)
      pltpu.sync_copy(tb, out_hbm.at[pl.ds(base, TS)])

    @pl.when(c == 0)
    def _():
      run(we_hbm, s_hbm)

    @pl.when(c == 1)
    def _():
      run(wg_hbm, y_hbm)

  return k(we, wg, srcp, dstp, dinv)


def kernel(adj, x, est_W1, est_b1, est_W2, est_b2,
           gnn_W1, gnn_b1, gnn_W2, gnn_b2):
  N, D = x.shape
  E = adj.shape[1]

  src = adj[0].astype(jnp.int32)
  dst = adj[1].astype(jnp.int32)

  EPW = -(-E // NW)
  CPW = _round_up(-(-EPW // CH), 4)
  EP = CPW * CH
  M = _round_up(N + 8, NSUB * 64)

  def padw(a, fill):
    a = jnp.pad(a, (0, NW * EPW - E), constant_values=fill).reshape(NW, EPW)
    a = jnp.pad(a, ((0, 0), (0, EP - EPW)), constant_values=fill)
    return a.reshape(NW, CPW, CH)

  srcp = padw(src, 0)
  dstp = padw(dst, N)

  degp = _deg_call(dstp, M, CPW)
  xs, dinv = _scale_call(degp, x, M)
  pp = _prop_call(xs, srcp, dstp, M, CPW)
  we, wg = _dense_call(pp, dinv, est_W1, est_b1, est_W2, est_b2,
                       gnn_W1, gnn_b1, gnn_W2, gnn_b2, N)
  sflat, yflat = _final_call(we.reshape(N), wg.reshape(N),
                             srcp, dstp, dinv.reshape(M), M, CPW)

  s = sflat[:N].reshape(N, 1)
  y = yflat[:N].reshape(N, 1)
  return (y, s)


# speedup vs baseline: 20.7799x; 1.0470x over previous
import functools

import jax
import jax.numpy as jnp
from jax import lax
from jax.experimental import pallas as pl
from jax.experimental.pallas import tpu as pltpu
from jax.experimental.pallas import tpu_sc as plsc

L = 16
NSUB = 16
NCORE = 2
NW = NCORE * NSUB
CH = 128


def _round_up(a, b):
  return (a + b - 1) // b * b


def _fill_flat(buf, nwords, value):
  v = jnp.full((L,), value, jnp.float32)

  def body(i, _):
    buf[pl.ds(pl.multiple_of(i * L, L), L)] = v
    return 0

  lax.fori_loop(0, nwords // L, body, 0)


def _zero_rows(buf, rows, cols):
  z = jnp.zeros((L,), jnp.float32)
  per_row = cols // L

  def body(i, _):
    r = i // per_row
    c = i % per_row
    buf[r, pl.ds(pl.multiple_of(c * L, L), L)] = z
    return 0

  lax.fori_loop(0, rows * per_row, body, 0)


def _deg_call(dstp, M, CPW):
  TS = M // NSUB
  mesh = plsc.VectorSubcoreMesh(core_axis_name="c", subcore_axis_name="s")

  @functools.partial(
      pl.kernel,
      out_type=jax.ShapeDtypeStruct((NCORE, M), jnp.float32),
      mesh=mesh,
      scratch_types=[
          pltpu.VMEM_SHARED((M,), jnp.float32),
          pltpu.VMEM((CPW, CH), jnp.int32),
          pltpu.VMEM((CH,), jnp.float32),
          pltpu.VMEM((TS,), jnp.float32),
      ],
  )
  def k(dstp_hbm, deg_hbm, acc, dstv, vones, zbuf):
    c = lax.axis_index("c")
    s = lax.axis_index("s")
    base = pl.multiple_of(s * TS, 64)
    _fill_flat(zbuf, TS, 0.0)
    pltpu.sync_copy(zbuf, acc.at[pl.ds(base, TS)])
    _fill_flat(vones, CH, 1.0)
    plsc.subcore_barrier()

    w = c * NSUB + s
    pltpu.sync_copy(dstp_hbm.at[w], dstv)

    def chunk(j, _):
      pltpu.sync_copy(vones, acc.at[dstv.at[j]], add=True)
      return 0

    lax.fori_loop(0, CPW, chunk, 0)
    plsc.subcore_barrier()
    pltpu.sync_copy(acc.at[pl.ds(base, TS)], zbuf)
    pltpu.sync_copy(zbuf, deg_hbm.at[c, pl.ds(base, TS)])

  return k(dstp)


def _scale_call(degp, x, M):
  N, D = x.shape

  def body(d0_ref, d1_ref, x_ref, xs_ref, dinv_ref):
    deg = d0_ref[...] + d1_ref[...]
    dinv = lax.rsqrt(jnp.maximum(deg, 1.0))
    dinv_ref[...] = dinv
    xs_ref[...] = x_ref[...] * dinv[:N]

  return pl.pallas_call(
      body,
      out_shape=[
          jax.ShapeDtypeStruct((N, D), jnp.float32),
          jax.ShapeDtypeStruct((M, 1), jnp.float32),
      ],
  )(degp[0].reshape(M, 1), degp[1].reshape(M, 1), x)


def _prop_call(xs, srcp, dstp, M, CPW):
  N, D = xs.shape
  TS = M // NSUB
  NB = 4
  CH3 = CH // 2
  CPW3 = 2 * CPW
  HCP = CPW3 // 4
  NT = HCP // NB
  mesh = plsc.VectorSubcoreMesh(core_axis_name="c", subcore_axis_name="s")

  srcp3 = srcp.reshape(NW, CPW3, CH3)
  dstp3 = dstp.reshape(NW, CPW3, CH3)

  @functools.partial(
      pl.kernel,
      out_type=jax.ShapeDtypeStruct((NCORE, M, D), jnp.float32),
      mesh=mesh,
      scratch_types=[
          pltpu.VMEM_SHARED((M, D), jnp.float32),
          pltpu.VMEM((HCP, CH3), jnp.int32),
          pltpu.VMEM((HCP, CH3), jnp.int32),
          [pltpu.VMEM((CH3, D), jnp.float32) for _ in range(NB)],
          [pltpu.SemaphoreType.DMA for _ in range(NB)],
          [pltpu.SemaphoreType.DMA for _ in range(NB)],
      ],
  )
  def k(xs_hbm, srcp_hbm, dstp_hbm, pp_hbm,
        acc, srcv, dstv, rv, gs, ss):
    c = lax.axis_index("c")
    s = lax.axis_index("s")
    base = pl.multiple_of(s * TS, 64)
    _zero_rows(rv[0], CH3, D)

    def zc(kk, _):
      pltpu.sync_copy(rv[0], acc.at[pl.ds(base + kk * CH3, CH3)])
      return 0

    lax.fori_loop(0, TS // CH3, zc, 0)
    plsc.subcore_barrier()

    w = c * NSUB + s

    def gather(b, j):
      pltpu.async_copy(xs_hbm.at[srcv.at[j]], rv[b], gs[b])

    def scat(b, j):
      pltpu.async_copy(rv[b], acc.at[dstv.at[j]], ss[b], add=True)

    def wait_g(b):
      pltpu.make_async_copy(xs_hbm.at[srcv.at[0]], rv[b], gs[b]).wait()

    def wait_s(b):
      pltpu.make_async_copy(rv[b], acc.at[dstv.at[0]], ss[b]).wait()

    def ring(t, _):
      j = NB * t

      @pl.when(t > 0)
      def _():
        wait_s(2)
        gather(2, j + 2)
        wait_s(3)
        gather(3, j + 3)

      @pl.when(t == 0)
      def _():
        gather(2, 2)
        gather(3, 3)

      wait_g(0)
      scat(0, j)
      wait_g(1)
      scat(1, j + 1)

      @pl.when(t + 1 < NT)
      def _():
        wait_s(0)
        gather(0, j + 4)
        wait_s(1)
        gather(1, j + 5)

      @pl.when(t + 1 == NT)
      def _():
        wait_s(0)
        wait_s(1)

      wait_g(2)
      scat(2, j + 2)
      wait_g(3)
      scat(3, j + 3)

      @pl.when(t + 1 == NT)
      def _():
        wait_s(2)
        wait_s(3)

      return 0

    for h in range(4):
      pltpu.sync_copy(srcp_hbm.at[w, pl.ds(h * HCP, HCP)], srcv)
      pltpu.sync_copy(dstp_hbm.at[w, pl.ds(h * HCP, HCP)], dstv)
      gather(0, 0)
      gather(1, 1)
      lax.fori_loop(0, NT, ring, 0)

    plsc.subcore_barrier()

    def oc(kk, _):
      rbase = pl.multiple_of(base + kk * CH3, 64)
      pltpu.sync_copy(acc.at[pl.ds(rbase, CH3)], rv[0])
      pltpu.sync_copy(rv[0], pp_hbm.at[c, pl.ds(rbase, CH3)])
      return 0

    lax.fori_loop(0, TS // CH3, oc, 0)

  return k(xs, srcp3, dstp3)


def _dense_call(pp, dinv, eW1, eb1, eW2, eb2, gW1, gb1, gW2, gb2, N):
  D = pp.shape[2]
  H = eW1.shape[1]
  BR = 2000 if N % 2000 == 0 else N
  grid = (N // BR,)

  def body(p0_ref, p1_ref, dinv_ref, eW1_ref, eb1_ref, eW2_ref, eb2_ref,
           gW1_ref, gb1_ref, gW2_ref, gb2_ref, we_ref, wg_ref):
    dv = dinv_ref[...]
    p = (p0_ref[...] + p1_ref[...]) * dv

    he = jnp.maximum(
        jnp.dot(p, eW1_ref[...], preferred_element_type=jnp.float32)
        + eb1_ref[...], 0.0)
    we_ref[...] = (
        jnp.dot(he, eW2_ref[...], preferred_element_type=jnp.float32)
        + eb2_ref[...]) * dv

    hg = jnp.maximum(
        jnp.dot(p, gW1_ref[...], preferred_element_type=jnp.float32)
        + gb1_ref[...], 0.0)
    wg_ref[...] = (
        jnp.dot(hg, gW2_ref[...], preferred_element_type=jnp.float32)
        + gb2_ref[...]) * dv

  row_spec = pl.BlockSpec((BR, D), lambda i: (i, 0))
  col_spec = pl.BlockSpec((BR, 1), lambda i: (i, 0))
  w1_spec = pl.BlockSpec((D, H), lambda i: (0, 0))
  b1_spec = pl.BlockSpec((1, H), lambda i: (0, 0))
  w2_spec = pl.BlockSpec((H, 1), lambda i: (0, 0))
  b2_spec = pl.BlockSpec((1, 1), lambda i: (0, 0))

  return pl.pallas_call(
      body,
      grid=grid,
      in_specs=[row_spec, row_spec, col_spec,
                w1_spec, b1_spec, w2_spec, b2_spec,
                w1_spec, b1_spec, w2_spec, b2_spec],
      out_specs=[col_spec, col_spec],
      out_shape=[
          jax.ShapeDtypeStruct((N, 1), jnp.float32),
          jax.ShapeDtypeStruct((N, 1), jnp.float32),
      ],
  )(pp[0], pp[1], dinv,
    eW1, eb1.reshape(1, H), eW2, eb2.reshape(1, 1),
    gW1, gb1.reshape(1, H), gW2, gb2.reshape(1, 1))


def _final_call(we, wg, srcp, dstp, dinv, M, CPW):
  TS = M // NSUB
  mesh = plsc.VectorSubcoreMesh(core_axis_name="c", subcore_axis_name="s")

  @functools.partial(
      pl.kernel,
      out_type=[
          jax.ShapeDtypeStruct((M,), jnp.float32),
          jax.ShapeDtypeStruct((M,), jnp.float32),
      ],
      mesh=mesh,
      scratch_types=[
          pltpu.VMEM_SHARED((M,), jnp.float32),
          pltpu.VMEM(((NW // NSUB) * CPW, CH), jnp.int32),
          pltpu.VMEM(((NW // NSUB) * CPW, CH), jnp.int32),
          pltpu.VMEM((CH,), jnp.float32),
          pltpu.VMEM((CH,), jnp.float32),
          pltpu.VMEM((TS,), jnp.float32),
          pltpu.VMEM((TS,), jnp.float32),
          pltpu.SemaphoreType.DMA,
          pltpu.SemaphoreType.DMA,
          pltpu.SemaphoreType.DMA,
          pltpu.SemaphoreType.DMA,
      ],
  )
  def k(we_hbm, wg_hbm, srcp_hbm, dstp_hbm, dinv_hbm, s_hbm, y_hbm,
        acc, srcv, dstv, vb0, vb1, tb, db, gs0, gs1, ss0, ss1):
    c = lax.axis_index("c")
    s = lax.axis_index("s")
    base = pl.multiple_of(s * TS, 64)
    WPT = NW // NSUB
    NT = (WPT * CPW) // 2
    sv = srcv
    dv = dstv

    def run(w_hbm, out_hbm):
      _fill_flat(tb, TS, 0.0)
      pltpu.sync_copy(tb, acc.at[pl.ds(base, TS)])
      plsc.subcore_barrier()

      for r in range(WPT):
        pltpu.sync_copy(srcp_hbm.at[s * WPT + r],
                        srcv.at[pl.ds(r * CPW, CPW)])
        pltpu.sync_copy(dstp_hbm.at[s * WPT + r],
                        dstv.at[pl.ds(r * CPW, CPW)])

      def wait_g(vb, gs):
        pltpu.make_async_copy(w_hbm.at[sv.at[0]], vb, gs).wait()

      def wait_s(vb, ss):
        pltpu.make_async_copy(vb, acc.at[dv.at[0]], ss).wait()

      def chunk2(t, _):
        wait_g(vb0, gs0)
        pltpu.async_copy(vb0, acc.at[dv.at[2 * t]], ss0, add=True)
        wait_g(vb1, gs1)
        pltpu.async_copy(vb1, acc.at[dv.at[2 * t + 1]], ss1, add=True)

        @pl.when(t + 1 < NT)
        def _():
          wait_s(vb0, ss0)
          pltpu.async_copy(w_hbm.at[sv.at[2 * t + 2]], vb0, gs0)
          wait_s(vb1, ss1)
          pltpu.async_copy(w_hbm.at[sv.at[2 * t + 3]], vb1, gs1)

        @pl.when(t + 1 == NT)
        def _():
          wait_s(vb0, ss0)
          wait_s(vb1, ss1)

        return 0

      pltpu.async_copy(w_hbm.at[sv.at[0]], vb0, gs0)
      pltpu.async_copy(w_hbm.at[sv.at[1]], vb1, gs1)
      lax.fori_loop(0, NT, chunk2, 0)

      plsc.subcore_barrier()
      pltpu.sync_copy(dinv_hbm.at[pl.ds(base, TS)], db)
      pltpu.sync_copy(acc.at[pl.ds(base, TS)], tb)

      def mul(i, _):
        o = pl.multiple_of(i * L, L)
        tb[pl.ds(o, L)] = tb[pl.ds(o, L)] * db[pl.ds(o, L)]
        return 0

      lax.fori_loop(0, TS // L, mul, 0)
      pltpu.sync_copy(tb, out_hbm.at[pl.ds(base, TS)])

    @pl.when(c == 0)
    def _():
      run(we_hbm, s_hbm)

    @pl.when(c == 1)
    def _():
      run(wg_hbm, y_hbm)

  return k(we, wg, srcp, dstp, dinv)


def kernel(adj, x, est_W1, est_b1, est_W2, est_b2,
           gnn_W1, gnn_b1, gnn_W2, gnn_b2):
  N, D = x.shape
  E = adj.shape[1]

  src = adj[0].astype(jnp.int32)
  dst = adj[1].astype(jnp.int32)

  EPW = -(-E // NW)
  CPW = _round_up(-(-EPW // CH), 8)
  EP = CPW * CH
  M = _round_up(N + 8, NSUB * 64)

  def padw(a, fill):
    a = jnp.pad(a, (0, NW * EPW - E), constant_values=fill).reshape(NW, EPW)
    a = jnp.pad(a, ((0, 0), (0, EP - EPW)), constant_values=fill)
    return a.reshape(NW, CPW, CH)

  srcp = padw(src, 0)
  dstp = padw(dst, N)

  degp = _deg_call(dstp, M, CPW)
  xs, dinv = _scale_call(degp, x, M)
  pp = _prop_call(xs, srcp, dstp, M, CPW)
  we, wg = _dense_call(pp, dinv, est_W1, est_b1, est_W2, est_b2,
                       gnn_W1, gnn_b1, gnn_W2, gnn_b2, N)
  sflat, yflat = _final_call(we.reshape(N), wg.reshape(N),
                             srcp, dstp, dinv.reshape(M), M, CPW)

  s = sflat[:N].reshape(N, 1)
  y = yflat[:N].reshape(N, 1)
  return (y, s)


# speedup vs baseline: 21.4661x; 1.0330x over previous
import functools

import jax
import jax.numpy as jnp
from jax import lax
from jax.experimental import pallas as pl
from jax.experimental.pallas import tpu as pltpu
from jax.experimental.pallas import tpu_sc as plsc

L = 16
NSUB = 16
NCORE = 2
NW = NCORE * NSUB
CH = 128


def _round_up(a, b):
  return (a + b - 1) // b * b


def _fill_flat(buf, nwords, value):
  v = jnp.full((L,), value, jnp.float32)

  def body(i, _):
    buf[pl.ds(pl.multiple_of(i * L, L), L)] = v
    return 0

  lax.fori_loop(0, nwords // L, body, 0)


def _zero_rows(buf, rows, cols):
  z = jnp.zeros((L,), jnp.float32)
  per_row = cols // L

  def body(i, _):
    r = i // per_row
    c = i % per_row
    buf[r, pl.ds(pl.multiple_of(c * L, L), L)] = z
    return 0

  lax.fori_loop(0, rows * per_row, body, 0)


def _deg_call(dstp, M, CPW):
  TS = M // NSUB
  mesh = plsc.VectorSubcoreMesh(core_axis_name="c", subcore_axis_name="s")

  @functools.partial(
      pl.kernel,
      out_type=jax.ShapeDtypeStruct((NCORE, M), jnp.float32),
      mesh=mesh,
      scratch_types=[
          pltpu.VMEM_SHARED((M,), jnp.float32),
          pltpu.VMEM((CPW, CH), jnp.int32),
          pltpu.VMEM((CH,), jnp.float32),
          pltpu.VMEM((TS,), jnp.float32),
          [pltpu.SemaphoreType.DMA for _ in range(4)],
      ],
  )
  def k(dstp_hbm, deg_hbm, acc, dstv, vones, zbuf, ss):
    c = lax.axis_index("c")
    s = lax.axis_index("s")
    base = pl.multiple_of(s * TS, 64)
    _fill_flat(zbuf, TS, 0.0)
    pltpu.sync_copy(zbuf, acc.at[pl.ds(base, TS)])
    _fill_flat(vones, CH, 1.0)
    plsc.subcore_barrier()

    w = c * NSUB + s
    pltpu.sync_copy(dstp_hbm.at[w], dstv)

    def wait_s(b):
      pltpu.make_async_copy(vones, acc.at[dstv.at[0]], ss[b]).wait()

    def ring(t, _):
      for b in range(4):
        @pl.when(t > 0)
        def _():
          wait_s(b)

        pltpu.async_copy(vones, acc.at[dstv.at[4 * t + b]], ss[b], add=True)
      return 0

    lax.fori_loop(0, CPW // 4, ring, 0)
    for b in range(4):
      wait_s(b)
    plsc.subcore_barrier()
    pltpu.sync_copy(acc.at[pl.ds(base, TS)], zbuf)
    pltpu.sync_copy(zbuf, deg_hbm.at[c, pl.ds(base, TS)])

  return k(dstp)


def _scale_call(degp, x, M):
  N, D = x.shape

  def body(d0_ref, d1_ref, x_ref, xs_ref, dinv_ref):
    deg = d0_ref[...] + d1_ref[...]
    dinv = lax.rsqrt(jnp.maximum(deg, 1.0))
    dinv_ref[...] = dinv
    xs_ref[...] = x_ref[...] * dinv[:N]

  return pl.pallas_call(
      body,
      out_shape=[
          jax.ShapeDtypeStruct((N, D), jnp.float32),
          jax.ShapeDtypeStruct((M, 1), jnp.float32),
      ],
  )(degp[0].reshape(M, 1), degp[1].reshape(M, 1), x)


def _prop_call(xs, srcp, dstp, M, CPW):
  N, D = xs.shape
  TS = M // NSUB
  NB = 4
  CH3 = CH // 2
  CPW3 = 2 * CPW
  HCP = CPW3 // 4
  NT = HCP // NB
  mesh = plsc.VectorSubcoreMesh(core_axis_name="c", subcore_axis_name="s")

  srcp3 = srcp.reshape(NW, CPW3, CH3)
  dstp3 = dstp.reshape(NW, CPW3, CH3)

  @functools.partial(
      pl.kernel,
      out_type=jax.ShapeDtypeStruct((NCORE, M, D), jnp.float32),
      mesh=mesh,
      scratch_types=[
          pltpu.VMEM_SHARED((M, D), jnp.float32),
          pltpu.VMEM((HCP, CH3), jnp.int32),
          pltpu.VMEM((HCP, CH3), jnp.int32),
          [pltpu.VMEM((CH3, D), jnp.float32) for _ in range(NB)],
          [pltpu.SemaphoreType.DMA for _ in range(NB)],
          [pltpu.SemaphoreType.DMA for _ in range(NB)],
      ],
  )
  def k(xs_hbm, srcp_hbm, dstp_hbm, pp_hbm,
        acc, srcv, dstv, rv, gs, ss):
    c = lax.axis_index("c")
    s = lax.axis_index("s")
    base = pl.multiple_of(s * TS, 64)
    _zero_rows(rv[0], CH3, D)

    def zc(kk, _):
      pltpu.sync_copy(rv[0], acc.at[pl.ds(base + kk * CH3, CH3)])
      return 0

    lax.fori_loop(0, TS // CH3, zc, 0)
    plsc.subcore_barrier()

    w = c * NSUB + s

    def gather(b, j):
      pltpu.async_copy(xs_hbm.at[srcv.at[j]], rv[b], gs[b])

    def scat(b, j):
      pltpu.async_copy(rv[b], acc.at[dstv.at[j]], ss[b], add=True)

    def wait_g(b):
      pltpu.make_async_copy(xs_hbm.at[srcv.at[0]], rv[b], gs[b]).wait()

    def wait_s(b):
      pltpu.make_async_copy(rv[b], acc.at[dstv.at[0]], ss[b]).wait()

    def ring(t, _):
      j = NB * t

      @pl.when(t > 0)
      def _():
        wait_s(2)
        gather(2, j + 2)
        wait_s(3)
        gather(3, j + 3)

      @pl.when(t == 0)
      def _():
        gather(2, 2)
        gather(3, 3)

      wait_g(0)
      scat(0, j)
      wait_g(1)
      scat(1, j + 1)

      @pl.when(t + 1 < NT)
      def _():
        wait_s(0)
        gather(0, j + 4)
        wait_s(1)
        gather(1, j + 5)

      @pl.when(t + 1 == NT)
      def _():
        wait_s(0)
        wait_s(1)

      wait_g(2)
      scat(2, j + 2)
      wait_g(3)
      scat(3, j + 3)

      @pl.when(t + 1 == NT)
      def _():
        wait_s(2)
        wait_s(3)

      return 0

    for h in range(4):
      pltpu.sync_copy(srcp_hbm.at[w, pl.ds(h * HCP, HCP)], srcv)
      pltpu.sync_copy(dstp_hbm.at[w, pl.ds(h * HCP, HCP)], dstv)
      gather(0, 0)
      gather(1, 1)
      lax.fori_loop(0, NT, ring, 0)

    plsc.subcore_barrier()

    def oc(kk, _):
      rbase = pl.multiple_of(base + kk * CH3, 64)
      pltpu.sync_copy(acc.at[pl.ds(rbase, CH3)], rv[0])
      pltpu.sync_copy(rv[0], pp_hbm.at[c, pl.ds(rbase, CH3)])
      return 0

    lax.fori_loop(0, TS // CH3, oc, 0)

  return k(xs, srcp3, dstp3)


def _dense_call(pp, dinv, eW1, eb1, eW2, eb2, gW1, gb1, gW2, gb2, N):
  D = pp.shape[2]
  H = eW1.shape[1]
  BR = 2000 if N % 2000 == 0 else N
  grid = (N // BR,)

  def body(p0_ref, p1_ref, dinv_ref, eW1_ref, eb1_ref, eW2_ref, eb2_ref,
           gW1_ref, gb1_ref, gW2_ref, gb2_ref, we_ref, wg_ref):
    dv = dinv_ref[...]
    p = (p0_ref[...] + p1_ref[...]) * dv

    he = jnp.maximum(
        jnp.dot(p, eW1_ref[...], preferred_element_type=jnp.float32)
        + eb1_ref[...], 0.0)
    we_ref[...] = (
        jnp.dot(he, eW2_ref[...], preferred_element_type=jnp.float32)
        + eb2_ref[...]) * dv

    hg = jnp.maximum(
        jnp.dot(p, gW1_ref[...], preferred_element_type=jnp.float32)
        + gb1_ref[...], 0.0)
    wg_ref[...] = (
        jnp.dot(hg, gW2_ref[...], preferred_element_type=jnp.float32)
        + gb2_ref[...]) * dv

  row_spec = pl.BlockSpec((BR, D), lambda i: (i, 0))
  col_spec = pl.BlockSpec((BR, 1), lambda i: (i, 0))
  w1_spec = pl.BlockSpec((D, H), lambda i: (0, 0))
  b1_spec = pl.BlockSpec((1, H), lambda i: (0, 0))
  w2_spec = pl.BlockSpec((H, 1), lambda i: (0, 0))
  b2_spec = pl.BlockSpec((1, 1), lambda i: (0, 0))

  return pl.pallas_call(
      body,
      grid=grid,
      in_specs=[row_spec, row_spec, col_spec,
                w1_spec, b1_spec, w2_spec, b2_spec,
                w1_spec, b1_spec, w2_spec, b2_spec],
      out_specs=[col_spec, col_spec],
      out_shape=[
          jax.ShapeDtypeStruct((N, 1), jnp.float32),
          jax.ShapeDtypeStruct((N, 1), jnp.float32),
      ],
  )(pp[0], pp[1], dinv,
    eW1, eb1.reshape(1, H), eW2, eb2.reshape(1, 1),
    gW1, gb1.reshape(1, H), gW2, gb2.reshape(1, 1))


def _final_call(we, wg, srcp, dstp, dinv, M, CPW):
  TS = M // NSUB
  WPT = NW // NSUB
  NB = 4
  NTOT = WPT * CPW
  NT = NTOT // NB
  mesh = plsc.VectorSubcoreMesh(core_axis_name="c", subcore_axis_name="s")

  @functools.partial(
      pl.kernel,
      out_type=[
          jax.ShapeDtypeStruct((M,), jnp.float32),
          jax.ShapeDtypeStruct((M,), jnp.float32),
      ],
      mesh=mesh,
      scratch_types=[
          pltpu.VMEM_SHARED((M,), jnp.float32),
          pltpu.VMEM((WPT * CPW, CH), jnp.int32),
          pltpu.VMEM((WPT * CPW, CH), jnp.int32),
          [pltpu.VMEM((CH,), jnp.float32) for _ in range(NB)],
          pltpu.VMEM((TS,), jnp.float32),
          pltpu.VMEM((TS,), jnp.float32),
          [pltpu.SemaphoreType.DMA for _ in range(NB)],
          [pltpu.SemaphoreType.DMA for _ in range(NB)],
      ],
  )
  def k(we_hbm, wg_hbm, srcp_hbm, dstp_hbm, dinv_hbm, s_hbm, y_hbm,
        acc, srcv, dstv, vb, tb, db, gs, ss):
    c = lax.axis_index("c")
    s = lax.axis_index("s")
    base = pl.multiple_of(s * TS, 64)

    def run(w_hbm, out_hbm):
      _fill_flat(tb, TS, 0.0)
      pltpu.sync_copy(tb, acc.at[pl.ds(base, TS)])
      plsc.subcore_barrier()

      for r in range(WPT):
        pltpu.sync_copy(srcp_hbm.at[s * WPT + r],
                        srcv.at[pl.ds(r * CPW, CPW)])
        pltpu.sync_copy(dstp_hbm.at[s * WPT + r],
                        dstv.at[pl.ds(r * CPW, CPW)])

      def gather(b, j):
        pltpu.async_copy(w_hbm.at[srcv.at[j]], vb[b], gs[b])

      def scat(b, j):
        pltpu.async_copy(vb[b], acc.at[dstv.at[j]], ss[b], add=True)

      def wait_g(b):
        pltpu.make_async_copy(w_hbm.at[srcv.at[0]], vb[b], gs[b]).wait()

      def wait_s(b):
        pltpu.make_async_copy(vb[b], acc.at[dstv.at[0]], ss[b]).wait()

      def ring(t, _):
        j = NB * t

        @pl.when(t > 0)
        def _():
          wait_s(2)
          gather(2, j + 2)
          wait_s(3)
          gather(3, j + 3)

        @pl.when(t == 0)
        def _():
          gather(2, 2)
          gather(3, 3)

        wait_g(0)
        scat(0, j)
        wait_g(1)
        scat(1, j + 1)

        @pl.when(t + 1 < NT)
        def _():
          wait_s(0)
          gather(0, j + 4)
          wait_s(1)
          gather(1, j + 5)

        @pl.when(t + 1 == NT)
        def _():
          wait_s(0)
          wait_s(1)

        wait_g(2)
        scat(2, j + 2)
        wait_g(3)
        scat(3, j + 3)

        @pl.when(t + 1 == NT)
        def _():
          wait_s(2)
          wait_s(3)

        return 0

      gather(0, 0)
      gather(1, 1)
      lax.fori_loop(0, NT, ring, 0)

      plsc.subcore_barrier()
      pltpu.sync_copy(dinv_hbm.at[pl.ds(base, TS)], db)
      pltpu.sync_copy(acc.at[pl.ds(base, TS)], tb)

      def mul(i, _):
        o = pl.multiple_of(i * L, L)
        tb[pl.ds(o, L)] = tb[pl.ds(o, L)] * db[pl.ds(o, L)]
        return 0

      lax.fori_loop(0, TS // L, mul, 0)
      pltpu.sync_copy(tb, out_hbm.at[pl.ds(base, TS)])

    @pl.when(c == 0)
    def _():
      run(we_hbm, s_hbm)

    @pl.when(c == 1)
    def _():
      run(wg_hbm, y_hbm)

  return k(we, wg, srcp, dstp, dinv)


def kernel(adj, x, est_W1, est_b1, est_W2, est_b2,
           gnn_W1, gnn_b1, gnn_W2, gnn_b2):
  N, D = x.shape
  E = adj.shape[1]

  src = adj[0].astype(jnp.int32)
  dst = adj[1].astype(jnp.int32)

  EPW = -(-E // NW)
  CPW = _round_up(-(-EPW // CH), 8)
  EP = CPW * CH
  M = _round_up(N + 8, NSUB * 64)

  def padw(a, fill):
    a = jnp.pad(a, (0, NW * EPW - E), constant_values=fill).reshape(NW, EPW)
    a = jnp.pad(a, ((0, 0), (0, EP - EPW)), constant_values=fill)
    return a.reshape(NW, CPW, CH)

  srcp = padw(src, 0)
  dstp = padw(dst, N)

  degp = _deg_call(dstp, M, CPW)
  xs, dinv = _scale_call(degp, x, M)
  pp = _prop_call(xs, srcp, dstp, M, CPW)
  we, wg = _dense_call(pp, dinv, est_W1, est_b1, est_W2, est_b2,
                       gnn_W1, gnn_b1, gnn_W2, gnn_b2, N)
  wep = jnp.pad(we.reshape(N), (0, M - N))
  wgp = jnp.pad(wg.reshape(N), (0, M - N))
  sflat, yflat = _final_call(wep, wgp, srcp, dstp, dinv.reshape(M), M, CPW)

  s = sflat[:N].reshape(N, 1)
  y = yflat[:N].reshape(N, 1)
  return (y, s)


# speedup vs baseline: 24.5216x; 1.1423x over previous
import functools

import jax
import jax.numpy as jnp
from jax import lax
from jax.experimental import pallas as pl
from jax.experimental.pallas import tpu as pltpu
from jax.experimental.pallas import tpu_sc as plsc

L = 16
NSUB = 16
NCORE = 2
NW = NCORE * NSUB
CH = 128


def _round_up(a, b):
  return (a + b - 1) // b * b


def _fill_flat(buf, nwords, value):
  v = jnp.full((L,), value, jnp.float32)

  def body(i, _):
    buf[pl.ds(pl.multiple_of(i * L, L), L)] = v
    return 0

  lax.fori_loop(0, nwords // L, body, 0)


def _zero_rows(buf, rows, cols):
  z = jnp.zeros((L,), jnp.float32)
  per_row = cols // L

  def body(i, _):
    r = i // per_row
    c = i % per_row
    buf[r, pl.ds(pl.multiple_of(c * L, L), L)] = z
    return 0

  lax.fori_loop(0, rows * per_row, body, 0)


def _deg_call(dstp, M, CPW):
  TS = M // NSUB
  mesh = plsc.VectorSubcoreMesh(core_axis_name="c", subcore_axis_name="s")

  @functools.partial(
      pl.kernel,
      out_type=jax.ShapeDtypeStruct((NCORE, M), jnp.float32),
      mesh=mesh,
      scratch_types=[
          pltpu.VMEM_SHARED((M,), jnp.float32),
          pltpu.VMEM((CPW, CH), jnp.int32),
          pltpu.VMEM((CH,), jnp.float32),
          pltpu.VMEM((TS,), jnp.float32),
          [pltpu.SemaphoreType.DMA for _ in range(4)],
      ],
  )
  def k(dstp_hbm, deg_hbm, acc, dstv, vones, zbuf, ss):
    c = lax.axis_index("c")
    s = lax.axis_index("s")
    base = pl.multiple_of(s * TS, 64)
    _fill_flat(zbuf, TS, 0.0)
    pltpu.sync_copy(zbuf, acc.at[pl.ds(base, TS)])
    _fill_flat(vones, CH, 1.0)
    plsc.subcore_barrier()

    w = c * NSUB + s
    pltpu.sync_copy(dstp_hbm.at[w], dstv)

    def wait_s(b):
      pltpu.make_async_copy(vones, acc.at[dstv.at[0]], ss[b]).wait()

    def ring(t, _):
      for b in range(4):
        @pl.when(t > 0)
        def _():
          wait_s(b)

        pltpu.async_copy(vones, acc.at[dstv.at[4 * t + b]], ss[b], add=True)
      return 0

    lax.fori_loop(0, CPW // 4, ring, 0)
    for b in range(4):
      wait_s(b)
    plsc.subcore_barrier()
    pltpu.sync_copy(acc.at[pl.ds(base, TS)], zbuf)
    pltpu.sync_copy(zbuf, deg_hbm.at[c, pl.ds(base, TS)])

  return k(dstp)


def _scale_call(degp, x, M):
  N, D = x.shape

  def body(d0_ref, d1_ref, x_ref, xs_ref, dinv_ref):
    deg = d0_ref[...] + d1_ref[...]
    dinv = lax.rsqrt(jnp.maximum(deg, 1.0))
    dinv_ref[...] = dinv
    xs_ref[...] = x_ref[...] * dinv[:N]

  return pl.pallas_call(
      body,
      out_shape=[
          jax.ShapeDtypeStruct((N, D), jnp.float32),
          jax.ShapeDtypeStruct((M, 1), jnp.float32),
      ],
  )(degp[0].reshape(M, 1), degp[1].reshape(M, 1), x)


def _prop_call(xs, srcp, dstp, M, CPW):
  N, D = xs.shape
  TS = M // NSUB
  NB = 4
  CH3 = CH // 2
  CPW3 = 2 * CPW
  HCP = CPW3 // 4
  NT = HCP // NB
  mesh = plsc.VectorSubcoreMesh(core_axis_name="c", subcore_axis_name="s")

  srcp3 = srcp.reshape(NW, CPW3, CH3)
  dstp3 = dstp.reshape(NW, CPW3, CH3)

  @functools.partial(
      pl.kernel,
      out_type=jax.ShapeDtypeStruct((NCORE, M, D), jnp.float32),
      mesh=mesh,
      scratch_types=[
          pltpu.VMEM_SHARED((M, D), jnp.float32),
          pltpu.VMEM((HCP, CH3), jnp.int32),
          pltpu.VMEM((HCP, CH3), jnp.int32),
          [pltpu.VMEM((CH3, D), jnp.float32) for _ in range(NB)],
          [pltpu.SemaphoreType.DMA for _ in range(NB)],
          [pltpu.SemaphoreType.DMA for _ in range(NB)],
      ],
  )
  def k(xs_hbm, srcp_hbm, dstp_hbm, pp_hbm,
        acc, srcv, dstv, rv, gs, ss):
    c = lax.axis_index("c")
    s = lax.axis_index("s")
    base = pl.multiple_of(s * TS, 64)
    _zero_rows(rv[0], CH3, D)

    def zc(kk, _):
      pltpu.sync_copy(rv[0], acc.at[pl.ds(base + kk * CH3, CH3)])
      return 0

    lax.fori_loop(0, TS // CH3, zc, 0)
    plsc.subcore_barrier()

    w = c * NSUB + s

    def gather(b, j):
      pltpu.async_copy(xs_hbm.at[srcv.at[j]], rv[b], gs[b])

    def scat(b, j):
      pltpu.async_copy(rv[b], acc.at[dstv.at[j]], ss[b], add=True)

    def wait_g(b):
      pltpu.make_async_copy(xs_hbm.at[srcv.at[0]], rv[b], gs[b]).wait()

    def wait_s(b):
      pltpu.make_async_copy(rv[b], acc.at[dstv.at[0]], ss[b]).wait()

    def ring(t, _):
      j = NB * t

      @pl.when(t > 0)
      def _():
        wait_s(2)
        gather(2, j + 2)
        wait_s(3)
        gather(3, j + 3)

      @pl.when(t == 0)
      def _():
        gather(2, 2)
        gather(3, 3)

      wait_g(0)
      scat(0, j)
      wait_g(1)
      scat(1, j + 1)

      @pl.when(t + 1 < NT)
      def _():
        wait_s(0)
        gather(0, j + 4)
        wait_s(1)
        gather(1, j + 5)

      @pl.when(t + 1 == NT)
      def _():
        wait_s(0)
        wait_s(1)

      wait_g(2)
      scat(2, j + 2)
      wait_g(3)
      scat(3, j + 3)

      @pl.when(t + 1 == NT)
      def _():
        wait_s(2)
        wait_s(3)

      return 0

    for h in range(4):
      pltpu.sync_copy(srcp_hbm.at[w, pl.ds(h * HCP, HCP)], srcv)
      pltpu.sync_copy(dstp_hbm.at[w, pl.ds(h * HCP, HCP)], dstv)
      gather(0, 0)
      gather(1, 1)
      lax.fori_loop(0, NT, ring, 0)

    plsc.subcore_barrier()

    def oc(kk, _):
      rbase = pl.multiple_of(base + kk * CH3, 64)
      pltpu.sync_copy(acc.at[pl.ds(rbase, CH3)], rv[0])
      pltpu.sync_copy(rv[0], pp_hbm.at[c, pl.ds(rbase, CH3)])
      return 0

    lax.fori_loop(0, TS // CH3, oc, 0)

  return k(xs, srcp3, dstp3)


def _dense_call(pp, dinv, eW1, eb1, eW2, eb2, gW1, gb1, gW2, gb2, N, M):
  D = pp.shape[2]
  H = eW1.shape[1]
  BR = 2000 if N % 2000 == 0 else N
  grid = (N // BR,)

  def body(p0_ref, p1_ref, dinv_ref, eW1_ref, eb1_ref, eW2_ref, eb2_ref,
           gW1_ref, gb1_ref, gW2_ref, gb2_ref, we_ref, wg_ref):
    dv = dinv_ref[...]
    p = (p0_ref[...] + p1_ref[...]) * dv

    he = jnp.maximum(
        jnp.dot(p, eW1_ref[...], preferred_element_type=jnp.float32)
        + eb1_ref[...], 0.0)
    we_ref[...] = (
        jnp.dot(he, eW2_ref[...], preferred_element_type=jnp.float32)
        + eb2_ref[...]) * dv

    hg = jnp.maximum(
        jnp.dot(p, gW1_ref[...], preferred_element_type=jnp.float32)
        + gb1_ref[...], 0.0)
    wg_ref[...] = (
        jnp.dot(hg, gW2_ref[...], preferred_element_type=jnp.float32)
        + gb2_ref[...]) * dv

  row_spec = pl.BlockSpec((BR, D), lambda i: (i, 0))
  col_spec = pl.BlockSpec((BR, 1), lambda i: (i, 0))
  w1_spec = pl.BlockSpec((D, H), lambda i: (0, 0))
  b1_spec = pl.BlockSpec((1, H), lambda i: (0, 0))
  w2_spec = pl.BlockSpec((H, 1), lambda i: (0, 0))
  b2_spec = pl.BlockSpec((1, 1), lambda i: (0, 0))

  return pl.pallas_call(
      body,
      grid=grid,
      in_specs=[row_spec, row_spec, col_spec,
                w1_spec, b1_spec, w2_spec, b2_spec,
                w1_spec, b1_spec, w2_spec, b2_spec],
      out_specs=[col_spec, col_spec],
      out_shape=[
          jax.ShapeDtypeStruct((M, 1), jnp.float32),
          jax.ShapeDtypeStruct((M, 1), jnp.float32),
      ],
  )(pp[0], pp[1], dinv,
    eW1, eb1.reshape(1, H), eW2, eb2.reshape(1, 1),
    gW1, gb1.reshape(1, H), gW2, gb2.reshape(1, 1))


def _final_call(we, wg, srcp, dstp, dinv, M, CPW):
  TS = M // NSUB
  WPT = NW // NSUB
  NB = 8
  NH = NB // 2
  NTOT = WPT * CPW
  NT = NTOT // NB
  mesh = plsc.VectorSubcoreMesh(core_axis_name="c", subcore_axis_name="s")

  @functools.partial(
      pl.kernel,
      out_type=[
          jax.ShapeDtypeStruct((M,), jnp.float32),
          jax.ShapeDtypeStruct((M,), jnp.float32),
      ],
      mesh=mesh,
      scratch_types=[
          pltpu.VMEM_SHARED((M,), jnp.float32),
          pltpu.VMEM_SHARED((M,), jnp.float32),
          pltpu.VMEM((WPT * CPW, CH), jnp.int32),
          pltpu.VMEM((WPT * CPW, CH), jnp.int32),
          [pltpu.VMEM((CH,), jnp.float32) for _ in range(NB)],
          pltpu.VMEM((TS,), jnp.float32),
          pltpu.VMEM((TS,), jnp.float32),
          [pltpu.SemaphoreType.DMA for _ in range(NB)],
          [pltpu.SemaphoreType.DMA for _ in range(NB)],
      ],
  )
  def k(we_hbm, wg_hbm, srcp_hbm, dstp_hbm, dinv_hbm, s_hbm, y_hbm,
        acc, wsh, srcv, dstv, vb, tb, db, gs, ss):
    c = lax.axis_index("c")
    s = lax.axis_index("s")
    base = pl.multiple_of(s * TS, 64)

    def run(w_hbm, out_hbm):
      _fill_flat(tb, TS, 0.0)
      pltpu.sync_copy(tb, acc.at[pl.ds(base, TS)])
      pltpu.sync_copy(w_hbm.at[pl.ds(base, TS)], db)
      pltpu.sync_copy(db, wsh.at[pl.ds(base, TS)])
      plsc.subcore_barrier()

      for r in range(WPT):
        pltpu.sync_copy(srcp_hbm.at[s * WPT + r],
                        srcv.at[pl.ds(r * CPW, CPW)])
        pltpu.sync_copy(dstp_hbm.at[s * WPT + r],
                        dstv.at[pl.ds(r * CPW, CPW)])

      def gather(b, j):
        pltpu.async_copy(wsh.at[srcv.at[j]], vb[b], gs[b])

      def scat(b, j):
        pltpu.async_copy(vb[b], acc.at[dstv.at[j]], ss[b], add=True)

      def wait_g(b):
        pltpu.make_async_copy(wsh.at[srcv.at[0]], vb[b], gs[b]).wait()

      def wait_s(b):
        pltpu.make_async_copy(vb[b], acc.at[dstv.at[0]], ss[b]).wait()

      def ring(t, _):
        j = NB * t

        @pl.when(t > 0)
        def _():
          for b in range(NH, NB):
            wait_s(b)
            gather(b, j + b)

        @pl.when(t == 0)
        def _():
          for b in range(NH, NB):
            gather(b, b)

        for b in range(NH):
          wait_g(b)
          scat(b, j + b)

        @pl.when(t + 1 < NT)
        def _():
          for b in range(NH):
            wait_s(b)
            gather(b, j + NB + b)

        @pl.when(t + 1 == NT)
        def _():
          for b in range(NH):
            wait_s(b)

        for b in range(NH, NB):
          wait_g(b)
          scat(b, j + b)

        @pl.when(t + 1 == NT)
        def _():
          for b in range(NH, NB):
            wait_s(b)

        return 0

      for b in range(NH):
        gather(b, b)
      lax.fori_loop(0, NT, ring, 0)

      plsc.subcore_barrier()
      pltpu.sync_copy(dinv_hbm.at[pl.ds(base, TS)], db)
      pltpu.sync_copy(acc.at[pl.ds(base, TS)], tb)

      def mul(i, _):
        o = pl.multiple_of(i * L, L)
        tb[pl.ds(o, L)] = tb[pl.ds(o, L)] * db[pl.ds(o, L)]
        return 0

      lax.fori_loop(0, TS // L, mul, 0)
      pltpu.sync_copy(tb, out_hbm.at[pl.ds(base, TS)])

    @pl.when(c == 0)
    def _():
      run(we_hbm, s_hbm)

    @pl.when(c == 1)
    def _():
      run(wg_hbm, y_hbm)

  return k(we, wg, srcp, dstp, dinv)


def kernel(adj, x, est_W1, est_b1, est_W2, est_b2,
           gnn_W1, gnn_b1, gnn_W2, gnn_b2):
  N, D = x.shape
  E = adj.shape[1]

  src = adj[0].astype(jnp.int32)
  dst = adj[1].astype(jnp.int32)

  EPW = -(-E // NW)
  CPW = _round_up(-(-EPW // CH), 8)
  EP = CPW * CH
  M = _round_up(N + 8, NSUB * 64)

  def padw(a, fill):
    a = jnp.pad(a, (0, NW * EPW - E), constant_values=fill).reshape(NW, EPW)
    a = jnp.pad(a, ((0, 0), (0, EP - EPW)), constant_values=fill)
    return a.reshape(NW, CPW, CH)

  srcp = padw(src, 0)
  dstp = padw(dst, N)

  degp = _deg_call(dstp, M, CPW)
  xs, dinv = _scale_call(degp, x, M)
  pp = _prop_call(xs, srcp, dstp, M, CPW)
  we, wg = _dense_call(pp, dinv, est_W1, est_b1, est_W2, est_b2,
                       gnn_W1, gnn_b1, gnn_W2, gnn_b2, N, M)
  sflat, yflat = _final_call(we.reshape(M), wg.reshape(M),
                             srcp, dstp, dinv.reshape(M), M, CPW)

  s = sflat[:N].reshape(N, 1)
  y = yflat[:N].reshape(N, 1)
  return (y, s)


# speedup vs baseline: 24.6010x; 1.0032x over previous
import functools

import jax
import jax.numpy as jnp
from jax import lax
from jax.experimental import pallas as pl
from jax.experimental.pallas import tpu as pltpu
from jax.experimental.pallas import tpu_sc as plsc

L = 16
NSUB = 16
NCORE = 2
NW = NCORE * NSUB
CH = 128


def _round_up(a, b):
  return (a + b - 1) // b * b


def _fill_flat(buf, nwords, value):
  v = jnp.full((L,), value, jnp.float32)

  def body(i, _):
    buf[pl.ds(pl.multiple_of(i * L, L), L)] = v
    return 0

  lax.fori_loop(0, nwords // L, body, 0)


def _zero_rows(buf, rows, cols):
  z = jnp.zeros((L,), jnp.float32)
  per_row = cols // L

  def body(i, _):
    r = i // per_row
    c = i % per_row
    buf[r, pl.ds(pl.multiple_of(c * L, L), L)] = z
    return 0

  lax.fori_loop(0, rows * per_row, body, 0)


def _deg_call(dstp, M, CPW):
  TS = M // NSUB
  mesh = plsc.VectorSubcoreMesh(core_axis_name="c", subcore_axis_name="s")

  @functools.partial(
      pl.kernel,
      out_type=jax.ShapeDtypeStruct((NCORE, M), jnp.float32),
      mesh=mesh,
      scratch_types=[
          pltpu.VMEM_SHARED((M,), jnp.float32),
          pltpu.VMEM((CPW, CH), jnp.int32),
          pltpu.VMEM((CH,), jnp.float32),
          pltpu.VMEM((TS,), jnp.float32),
          [pltpu.SemaphoreType.DMA for _ in range(4)],
      ],
  )
  def k(dstp_hbm, deg_hbm, acc, dstv, vones, zbuf, ss):
    c = lax.axis_index("c")
    s = lax.axis_index("s")
    base = pl.multiple_of(s * TS, 64)
    _fill_flat(zbuf, TS, 0.0)
    pltpu.sync_copy(zbuf, acc.at[pl.ds(base, TS)])
    _fill_flat(vones, CH, 1.0)
    plsc.subcore_barrier()

    w = c * NSUB + s
    pltpu.sync_copy(dstp_hbm.at[w], dstv)

    def wait_s(b):
      pltpu.make_async_copy(vones, acc.at[dstv.at[0]], ss[b]).wait()

    def ring(t, _):
      for b in range(4):
        @pl.when(t > 0)
        def _():
          wait_s(b)

        pltpu.async_copy(vones, acc.at[dstv.at[4 * t + b]], ss[b], add=True)
      return 0

    lax.fori_loop(0, CPW // 4, ring, 0)
    for b in range(4):
      wait_s(b)
    plsc.subcore_barrier()
    pltpu.sync_copy(acc.at[pl.ds(base, TS)], zbuf)
    pltpu.sync_copy(zbuf, deg_hbm.at[c, pl.ds(base, TS)])

  return k(dstp)


def _scale_call(degp, x, M):
  N, D = x.shape

  def body(d0_ref, d1_ref, x_ref, xs_ref, dinv_ref):
    deg = d0_ref[...] + d1_ref[...]
    dinv = lax.rsqrt(jnp.maximum(deg, 1.0))
    dinv_ref[...] = dinv
    xs_ref[...] = x_ref[...] * dinv[:N]

  return pl.pallas_call(
      body,
      out_shape=[
          jax.ShapeDtypeStruct((N, D), jnp.float32),
          jax.ShapeDtypeStruct((M, 1), jnp.float32),
      ],
  )(degp[0].reshape(M, 1), degp[1].reshape(M, 1), x)


def _prop_call(xs, srcp, dstp, M, CPW):
  N, D = xs.shape
  TS = M // NSUB
  NB = 4
  CH3 = CH // 2
  CPW3 = 2 * CPW
  HCP = CPW3 // 4
  NT = HCP // NB
  mesh = plsc.VectorSubcoreMesh(core_axis_name="c", subcore_axis_name="s")

  srcp3 = srcp.reshape(NW, CPW3, CH3)
  dstp3 = dstp.reshape(NW, CPW3, CH3)

  @functools.partial(
      pl.kernel,
      out_type=jax.ShapeDtypeStruct((NCORE, M, D), jnp.float32),
      mesh=mesh,
      scratch_types=[
          pltpu.VMEM_SHARED((M, D), jnp.float32),
          pltpu.VMEM((HCP, CH3), jnp.int32),
          pltpu.VMEM((HCP, CH3), jnp.int32),
          [pltpu.VMEM((CH3, D), jnp.float32) for _ in range(NB)],
          [pltpu.SemaphoreType.DMA for _ in range(NB)],
          [pltpu.SemaphoreType.DMA for _ in range(NB)],
      ],
  )
  def k(xs_hbm, srcp_hbm, dstp_hbm, pp_hbm,
        acc, srcv, dstv, rv, gs, ss):
    c = lax.axis_index("c")
    s = lax.axis_index("s")
    base = pl.multiple_of(s * TS, 64)
    _zero_rows(rv[0], CH3, D)

    def zc(kk, _):
      pltpu.sync_copy(rv[0], acc.at[pl.ds(base + kk * CH3, CH3)])
      return 0

    lax.fori_loop(0, TS // CH3, zc, 0)
    plsc.subcore_barrier()

    w = c * NSUB + s

    def gather(b, j):
      pltpu.async_copy(xs_hbm.at[srcv.at[j]], rv[b], gs[b])

    def scat(b, j):
      pltpu.async_copy(rv[b], acc.at[dstv.at[j]], ss[b], add=True)

    def wait_g(b):
      pltpu.make_async_copy(xs_hbm.at[srcv.at[0]], rv[b], gs[b]).wait()

    def wait_s(b):
      pltpu.make_async_copy(rv[b], acc.at[dstv.at[0]], ss[b]).wait()

    def ring(t, _):
      j = NB * t

      @pl.when(t > 0)
      def _():
        wait_s(2)
        gather(2, j + 2)
        wait_s(3)
        gather(3, j + 3)

      @pl.when(t == 0)
      def _():
        gather(2, 2)
        gather(3, 3)

      wait_g(0)
      scat(0, j)
      wait_g(1)
      scat(1, j + 1)

      @pl.when(t + 1 < NT)
      def _():
        wait_s(0)
        gather(0, j + 4)
        wait_s(1)
        gather(1, j + 5)

      @pl.when(t + 1 == NT)
      def _():
        wait_s(0)
        wait_s(1)

      wait_g(2)
      scat(2, j + 2)
      wait_g(3)
      scat(3, j + 3)

      @pl.when(t + 1 == NT)
      def _():
        wait_s(2)
        wait_s(3)

      return 0

    for h in range(4):
      pltpu.sync_copy(srcp_hbm.at[w, pl.ds(h * HCP, HCP)], srcv)
      pltpu.sync_copy(dstp_hbm.at[w, pl.ds(h * HCP, HCP)], dstv)
      gather(0, 0)
      gather(1, 1)
      lax.fori_loop(0, NT, ring, 0)

    plsc.subcore_barrier()

    NO = TS // CH3

    def oc(kk, _):
      rbase = pl.multiple_of(base + 2 * kk * CH3, 64)

      @pl.when(kk > 0)
      def _():
        pltpu.make_async_copy(acc.at[pl.ds(base, CH3)],
                              pp_hbm.at[c, pl.ds(base, CH3)], gs[0]).wait()
        pltpu.make_async_copy(acc.at[pl.ds(base, CH3)],
                              pp_hbm.at[c, pl.ds(base, CH3)], gs[1]).wait()

      pltpu.async_copy(acc.at[pl.ds(rbase, CH3)],
                       pp_hbm.at[c, pl.ds(rbase, CH3)], gs[0])
      pltpu.async_copy(acc.at[pl.ds(rbase + CH3, CH3)],
                       pp_hbm.at[c, pl.ds(rbase + CH3, CH3)], gs[1])
      return 0

    lax.fori_loop(0, NO // 2, oc, 0)
    pltpu.make_async_copy(acc.at[pl.ds(base, CH3)],
                          pp_hbm.at[c, pl.ds(base, CH3)], gs[0]).wait()
    pltpu.make_async_copy(acc.at[pl.ds(base, CH3)],
                          pp_hbm.at[c, pl.ds(base, CH3)], gs[1]).wait()

  return k(xs, srcp3, dstp3)


def _dense_call(pp, dinv, eW1, eb1, eW2, eb2, gW1, gb1, gW2, gb2, N, M):
  D = pp.shape[2]
  H = eW1.shape[1]
  BR = 2000 if N % 2000 == 0 else N
  grid = (N // BR,)

  def body(p0_ref, p1_ref, dinv_ref, eW1_ref, eb1_ref, eW2_ref, eb2_ref,
           gW1_ref, gb1_ref, gW2_ref, gb2_ref, we_ref, wg_ref):
    dv = dinv_ref[...]
    p = (p0_ref[...] + p1_ref[...]) * dv

    he = jnp.maximum(
        jnp.dot(p, eW1_ref[...], preferred_element_type=jnp.float32)
        + eb1_ref[...], 0.0)
    we_ref[...] = (
        jnp.dot(he, eW2_ref[...], preferred_element_type=jnp.float32)
        + eb2_ref[...]) * dv

    hg = jnp.maximum(
        jnp.dot(p, gW1_ref[...], preferred_element_type=jnp.float32)
        + gb1_ref[...], 0.0)
    wg_ref[...] = (
        jnp.dot(hg, gW2_ref[...], preferred_element_type=jnp.float32)
        + gb2_ref[...]) * dv

  row_spec = pl.BlockSpec((BR, D), lambda i: (i, 0))
  col_spec = pl.BlockSpec((BR, 1), lambda i: (i, 0))
  w1_spec = pl.BlockSpec((D, H), lambda i: (0, 0))
  b1_spec = pl.BlockSpec((1, H), lambda i: (0, 0))
  w2_spec = pl.BlockSpec((H, 1), lambda i: (0, 0))
  b2_spec = pl.BlockSpec((1, 1), lambda i: (0, 0))

  return pl.pallas_call(
      body,
      grid=grid,
      in_specs=[row_spec, row_spec, col_spec,
                w1_spec, b1_spec, w2_spec, b2_spec,
                w1_spec, b1_spec, w2_spec, b2_spec],
      out_specs=[col_spec, col_spec],
      out_shape=[
          jax.ShapeDtypeStruct((M, 1), jnp.float32),
          jax.ShapeDtypeStruct((M, 1), jnp.float32),
      ],
  )(pp[0], pp[1], dinv,
    eW1, eb1.reshape(1, H), eW2, eb2.reshape(1, 1),
    gW1, gb1.reshape(1, H), gW2, gb2.reshape(1, 1))


def _final_call(we, wg, srcp, dstp, dinv, M, CPW):
  TS = M // NSUB
  WPT = NW // NSUB
  NB = 8
  NH = NB // 2
  NTOT = WPT * CPW
  NT = NTOT // NB
  mesh = plsc.VectorSubcoreMesh(core_axis_name="c", subcore_axis_name="s")

  @functools.partial(
      pl.kernel,
      out_type=[
          jax.ShapeDtypeStruct((M,), jnp.float32),
          jax.ShapeDtypeStruct((M,), jnp.float32),
      ],
      mesh=mesh,
      scratch_types=[
          pltpu.VMEM_SHARED((M,), jnp.float32),
          pltpu.VMEM_SHARED((M,), jnp.float32),
          pltpu.VMEM((WPT * CPW, CH), jnp.int32),
          pltpu.VMEM((WPT * CPW, CH), jnp.int32),
          [pltpu.VMEM((CH,), jnp.float32) for _ in range(NB)],
          pltpu.VMEM((TS,), jnp.float32),
          pltpu.VMEM((TS,), jnp.float32),
          [pltpu.SemaphoreType.DMA for _ in range(NB)],
          [pltpu.SemaphoreType.DMA for _ in range(NB)],
      ],
  )
  def k(we_hbm, wg_hbm, srcp_hbm, dstp_hbm, dinv_hbm, s_hbm, y_hbm,
        acc, wsh, srcv, dstv, vb, tb, db, gs, ss):
    c = lax.axis_index("c")
    s = lax.axis_index("s")
    base = pl.multiple_of(s * TS, 64)

    def run(w_hbm, out_hbm):
      _fill_flat(tb, TS, 0.0)
      pltpu.sync_copy(tb, acc.at[pl.ds(base, TS)])
      pltpu.sync_copy(w_hbm.at[pl.ds(base, TS)], db)
      pltpu.sync_copy(db, wsh.at[pl.ds(base, TS)])
      plsc.subcore_barrier()

      for r in range(WPT):
        pltpu.sync_copy(srcp_hbm.at[s * WPT + r],
                        srcv.at[pl.ds(r * CPW, CPW)])
        pltpu.sync_copy(dstp_hbm.at[s * WPT + r],
                        dstv.at[pl.ds(r * CPW, CPW)])

      def gather(b, j):
        pltpu.async_copy(wsh.at[srcv.at[j]], vb[b], gs[b])

      def scat(b, j):
        pltpu.async_copy(vb[b], acc.at[dstv.at[j]], ss[b], add=True)

      def wait_g(b):
        pltpu.make_async_copy(wsh.at[srcv.at[0]], vb[b], gs[b]).wait()

      def wait_s(b):
        pltpu.make_async_copy(vb[b], acc.at[dstv.at[0]], ss[b]).wait()

      def ring(t, _):
        j = NB * t

        @pl.when(t > 0)
        def _():
          for b in range(NH, NB):
            wait_s(b)
            gather(b, j + b)

        @pl.when(t == 0)
        def _():
          for b in range(NH, NB):
            gather(b, b)

        for b in range(NH):
          wait_g(b)
          scat(b, j + b)

        @pl.when(t + 1 < NT)
        def _():
          for b in range(NH):
            wait_s(b)
            gather(b, j + NB + b)

        @pl.when(t + 1 == NT)
        def _():
          for b in range(NH):
            wait_s(b)

        for b in range(NH, NB):
          wait_g(b)
          scat(b, j + b)

        @pl.when(t + 1 == NT)
        def _():
          for b in range(NH, NB):
            wait_s(b)

        return 0

      for b in range(NH):
        gather(b, b)
      lax.fori_loop(0, NT, ring, 0)

      plsc.subcore_barrier()
      pltpu.sync_copy(dinv_hbm.at[pl.ds(base, TS)], db)
      pltpu.sync_copy(acc.at[pl.ds(base, TS)], tb)

      def mul(i, _):
        o = pl.multiple_of(i * L, L)
        tb[pl.ds(o, L)] = tb[pl.ds(o, L)] * db[pl.ds(o, L)]
        return 0

      lax.fori_loop(0, TS // L, mul, 0)
      pltpu.sync_copy(tb, out_hbm.at[pl.ds(base, TS)])

    @pl.when(c == 0)
    def _():
      run(we_hbm, s_hbm)

    @pl.when(c == 1)
    def _():
      run(wg_hbm, y_hbm)

  return k(we, wg, srcp, dstp, dinv)


def kernel(adj, x, est_W1, est_b1, est_W2, est_b2,
           gnn_W1, gnn_b1, gnn_W2, gnn_b2):
  N, D = x.shape
  E = adj.shape[1]

  src = adj[0].astype(jnp.int32)
  dst = adj[1].astype(jnp.int32)

  EPW = -(-E // NW)
  CPW = _round_up(-(-EPW // CH), 8)
  EP = CPW * CH
  M = _round_up(N + 8, NSUB * 64)

  def padw(a, fill):
    a = jnp.pad(a, (0, NW * EPW - E), constant_values=fill).reshape(NW, EPW)
    a = jnp.pad(a, ((0, 0), (0, EP - EPW)), constant_values=fill)
    return a.reshape(NW, CPW, CH)

  srcp = padw(src, 0)
  dstp = padw(dst, N)

  degp = _deg_call(dstp, M, CPW)
  xs, dinv = _scale_call(degp, x, M)
  pp = _prop_call(xs, srcp, dstp, M, CPW)
  we, wg = _dense_call(pp, dinv, est_W1, est_b1, est_W2, est_b2,
                       gnn_W1, gnn_b1, gnn_W2, gnn_b2, N, M)
  sflat, yflat = _final_call(we.reshape(M), wg.reshape(M),
                             srcp, dstp, dinv.reshape(M), M, CPW)

  s = sflat[:N].reshape(N, 1)
  y = yflat[:N].reshape(N, 1)
  return (y, s)
